# Initial kernel scaffold; baseline (speedup 1.0000x reference)
import jax
import jax.numpy as jnp
from jax.experimental import pallas as pl


def kernel(features, edge_index, W1, b1, W2, b2, W3, b3):
    raise NotImplementedError("write your pallas kernel here")


# speedup vs baseline: 3.1097x; 3.1097x over previous
import functools

import jax
import jax.numpy as jnp
from jax import lax
from jax.experimental import pallas as pl
from jax.experimental.pallas import tpu as pltpu
from jax.experimental.pallas import tpu_sc as plsc

N = 10000
E = 160000
IN_DIM = 256
H1 = 256
H2 = 128

NC = 2
NS = 16
NW = NC * NS

F32 = jnp.float32


_DEG_EPT = E // NW
_DEG_FULL = _DEG_EPT // 16
_DEG_REM = _DEG_EPT - _DEG_FULL * 16

@functools.cache
def _make_sc_degrees():
    mesh = plsc.VectorSubcoreMesh(core_axis_name="c", subcore_axis_name="s",
                                  num_cores=NC, num_subcores=NS)
    return pl.kernel(
        _sc_degrees_body,
        out_type=jax.ShapeDtypeStruct((NW, 2, N), F32),
        mesh=mesh,
        compiler_params=pltpu.CompilerParams(needs_layout_passes=False),
        scratch_types=[
            pltpu.VMEM((_DEG_EPT + 16,), jnp.int32),
            pltpu.VMEM((_DEG_EPT + 16,), jnp.int32),
            pltpu.VMEM((N,), F32),
            pltpu.VMEM((N,), F32),
        ],
    )


def _sc_degrees_body(src_hbm, dst_hbm, out_hbm, src_v, dst_v, acc_o, acc_i):
    c = lax.axis_index("c")
    s = lax.axis_index("s")
    wid = s * NC + c
    base = wid * _DEG_EPT

    zeros16 = jnp.zeros((16,), F32)

    def zero_body(i, _):
        acc_o[pl.ds(i * 16, 16)] = zeros16
        acc_i[pl.ds(i * 16, 16)] = zeros16
        return 0

    lax.fori_loop(0, N // 16, zero_body, 0)

    pltpu.sync_copy(src_hbm.at[pl.ds(base, _DEG_EPT)],
                    src_v.at[pl.ds(0, _DEG_EPT)])
    pltpu.sync_copy(dst_hbm.at[pl.ds(base, _DEG_EPT)],
                    dst_v.at[pl.ds(0, _DEG_EPT)])

    ones16 = jnp.ones((16,), F32)

    def body(i, _):
        sv = src_v[pl.ds(i * 16, 16)]
        dv = dst_v[pl.ds(i * 16, 16)]
        plsc.addupdate_scatter(acc_o, [sv], ones16)
        plsc.addupdate_scatter(acc_i, [dv], ones16)
        return 0

    lax.fori_loop(0, _DEG_FULL, body, 0)

    if _DEG_REM:
        mask = lax.iota(jnp.int32, 16) < _DEG_REM
        sv = src_v[pl.ds(_DEG_FULL * 16, 16)]
        dv = dst_v[pl.ds(_DEG_FULL * 16, 16)]
        plsc.addupdate_scatter(acc_o, [sv], ones16, mask=mask)
        plsc.addupdate_scatter(acc_i, [dv], ones16, mask=mask)

    pltpu.sync_copy(acc_o, out_hbm.at[wid, 0])
    pltpu.sync_copy(acc_i, out_hbm.at[wid, 1])


_EPT = E // NS
_CH = 128
_FULL = _EPT // _CH
_TAIL = _EPT - _FULL * _CH

_RPT = 624
_RPT_REM = N - NS * _RPT


@functools.cache
def _make_sc_propagate():
    mesh = plsc.VectorSubcoreMesh(core_axis_name="c", subcore_axis_name="s",
                                  num_cores=NC, num_subcores=NS)
    return pl.kernel(
        _sc_propagate_body,
        out_type=jax.ShapeDtypeStruct((NC, N, H2), F32),
        mesh=mesh,
        compiler_params=pltpu.CompilerParams(needs_layout_passes=False),
        scratch_types=[
            pltpu.VMEM((_CH,), jnp.int32),
            pltpu.VMEM((_CH,), jnp.int32),
            pltpu.VMEM((_CH, H2), F32),
            pltpu.VMEM((_TAIL,), jnp.int32),
            pltpu.VMEM((_TAIL,), jnp.int32),
            pltpu.VMEM((_TAIL, H2), F32),
            pltpu.VMEM_SHARED((N, H2), F32),
            pltpu.SemaphoreType.DMA,
        ],
    )


def _sc_propagate_body(ha_hbm, hb_hbm, src_hbm, dst_hbm, zero_hbm, out_hbm,
                       src_v, dst_v, rows_v, src_t, dst_t, rows_t, acc, sem):
    c = lax.axis_index("c")
    s = lax.axis_index("s")

    pltpu.sync_copy(zero_hbm.at[pl.ds(s * _RPT, _RPT)],
                    acc.at[pl.ds(s * _RPT, _RPT)])

    @pl.when(s == NS - 1)
    def _():
        pltpu.sync_copy(zero_hbm.at[pl.ds(NS * _RPT, _RPT_REM)],
                        acc.at[pl.ds(NS * _RPT, _RPT_REM)])

    plsc.subcore_barrier()

    base0 = s * _EPT

    def chunk(base, sv, dv, rv):
        pltpu.sync_copy(src_hbm.at[pl.ds(base, sv.shape[0])], sv)
        pltpu.sync_copy(dst_hbm.at[pl.ds(base, dv.shape[0])], dv)

        @pl.when(c == 0)
        def _():
            pltpu.async_copy(ha_hbm.at[sv], rv, sem).wait()

        @pl.when(c == 1)
        def _():
            pltpu.async_copy(hb_hbm.at[sv], rv, sem).wait()

        pltpu.sync_copy(rv, acc.at[dv], add=True)

    def body(j, _):
        chunk(base0 + j * _CH, src_v, dst_v, rows_v)
        return 0

    lax.fori_loop(0, _FULL, body, 0)
    if _TAIL:
        chunk(base0 + _FULL * _CH, src_t, dst_t, rows_t)

    plsc.subcore_barrier()
    pltpu.sync_copy(acc.at[pl.ds(s * _RPT, _RPT)],
                    out_hbm.at[c].at[pl.ds(s * _RPT, _RPT)])

    @pl.when(s == NS - 1)
    def _():
        pltpu.sync_copy(acc.at[pl.ds(NS * _RPT, _RPT_REM)],
                        out_hbm.at[c].at[pl.ds(NS * _RPT, _RPT_REM)])


_BM = 1000


def _norms_body(degp_ref, out_ref):
    deg = jnp.sum(degp_ref[...], axis=0)
    out_ref[...] = lax.rsqrt(jnp.maximum(deg, 1.0))


def _tc_norms(degp):
    return pl.pallas_call(
        _norms_body,
        out_shape=jax.ShapeDtypeStruct((2, N), F32),
    )(degp)


def _scale_body(f_ref, on_ref, a_ref, b_ref):
    xs = f_ref[...] * on_ref[...]
    a_ref[...] = xs[:, :H2]
    b_ref[...] = xs[:, H2:]


def _tc_scale(features, onorm):
    grid = (N // _BM,)
    return pl.pallas_call(
        _scale_body,
        grid=grid,
        in_specs=[
            pl.BlockSpec((_BM, IN_DIM), lambda i: (i, 0)),
            pl.BlockSpec((_BM, 1), lambda i: (i, 0)),
        ],
        out_specs=[
            pl.BlockSpec((_BM, H2), lambda i: (i, 0)),
            pl.BlockSpec((_BM, H2), lambda i: (i, 0)),
        ],
        out_shape=[
            jax.ShapeDtypeStruct((N, H2), F32),
            jax.ShapeDtypeStruct((N, H2), F32),
        ],
        compiler_params=pltpu.CompilerParams(
            dimension_semantics=("parallel",)),
    )(features, onorm)


def _layer1_body(a0_ref, a1_ref, w_ref, b_ref, in_ref, on_ref, ha_ref, hb_ref):
    w = w_ref[...]
    t = jnp.dot(a0_ref[0], w[:H2, :], preferred_element_type=F32)
    t += jnp.dot(a1_ref[0], w[H2:, :], preferred_element_type=F32)
    t = t * in_ref[...] + b_ref[...]
    t = jnp.maximum(t, 0.0) * on_ref[...]
    ha_ref[...] = t[:, :H2]
    hb_ref[...] = t[:, H2:]


def _tc_layer1(agg, W1, b1r, inorm, onorm):
    grid = (N // _BM,)
    return pl.pallas_call(
        _layer1_body,
        grid=grid,
        in_specs=[
            pl.BlockSpec((1, _BM, H2), lambda i: (0, i, 0)),
            pl.BlockSpec((1, _BM, H2), lambda i: (1, i, 0)),
            pl.BlockSpec((IN_DIM, H1), lambda i: (0, 0)),
            pl.BlockSpec((1, H1), lambda i: (0, 0)),
            pl.BlockSpec((_BM, 1), lambda i: (i, 0)),
            pl.BlockSpec((_BM, 1), lambda i: (i, 0)),
        ],
        out_specs=[
            pl.BlockSpec((_BM, H2), lambda i: (i, 0)),
            pl.BlockSpec((_BM, H2), lambda i: (i, 0)),
        ],
        out_shape=[
            jax.ShapeDtypeStruct((N, H2), F32),
            jax.ShapeDtypeStruct((N, H2), F32),
        ],
        compiler_params=pltpu.CompilerParams(
            dimension_semantics=("parallel",)),
    )(agg, agg, W1, b1r, inorm, onorm)


def _z_body(a0_ref, a1_ref, w_ref, b_ref, in_ref, nz_ref, z_ref):
    w = w_ref[...]
    t = jnp.dot(a0_ref[0], w[:H2, :], preferred_element_type=F32)
    t += jnp.dot(a1_ref[0], w[H2:, :], preferred_element_type=F32)
    t = t * in_ref[...] + b_ref[...]
    z_ref[...] = t[:, :H2] + nz_ref[...] * jnp.exp(t[:, H2:])


def _tc_z(aggh, W23, b23r, inorm, noise):
    grid = (N // _BM,)
    return pl.pallas_call(
        _z_body,
        grid=grid,
        in_specs=[
            pl.BlockSpec((1, _BM, H2), lambda i: (0, i, 0)),
            pl.BlockSpec((1, _BM, H2), lambda i: (1, i, 0)),
            pl.BlockSpec((H1, 2 * H2), lambda i: (0, 0)),
            pl.BlockSpec((1, 2 * H2), lambda i: (0, 0)),
            pl.BlockSpec((_BM, 1), lambda i: (i, 0)),
            pl.BlockSpec((_BM, H2), lambda i: (i, 0)),
        ],
        out_specs=pl.BlockSpec((_BM, H2), lambda i: (i, 0)),
        out_shape=jax.ShapeDtypeStruct((N, H2), F32),
        compiler_params=pltpu.CompilerParams(
            dimension_semantics=("parallel",)),
    )(aggh, aggh, W23, b23r, inorm, noise)


_DBM = 512


def _dec_body(zr_ref, zc_ref, out_ref):
    acc = lax.dot_general(zr_ref[...], zc_ref[...],
                          (((1,), (1,)), ((), ())),
                          preferred_element_type=F32)
    out_ref[...] = jax.nn.sigmoid(acc)


def _tc_decoder(z):
    grid = (pl.cdiv(N, _DBM), pl.cdiv(N, _DBM))
    return pl.pallas_call(
        _dec_body,
        grid=grid,
        in_specs=[
            pl.BlockSpec((_DBM, H2), lambda i, j: (i, 0)),
            pl.BlockSpec((_DBM, H2), lambda i, j: (j, 0)),
        ],
        out_specs=pl.BlockSpec((_DBM, _DBM), lambda i, j: (i, j)),
        out_shape=jax.ShapeDtypeStruct((N, N), F32),
        compiler_params=pltpu.CompilerParams(
            dimension_semantics=("parallel", "parallel")),
    )(z, z)


def kernel(features, edge_index, W1, b1, W2, b2, W3, b3):
    src = edge_index[0]
    dst = edge_index[1]

    degp = _make_sc_degrees()(src, dst)
    norms = _tc_norms(degp)
    onorm = norms[0].reshape(N, 1)
    inorm = norms[1].reshape(N, 1)

    xsA, xsB = _tc_scale(features, onorm)
    zeros = jnp.zeros((N, H2), F32)
    propagate = _make_sc_propagate()
    agg = propagate(xsA, xsB, src, dst, zeros)
    hA, hB = _tc_layer1(agg, W1, b1.reshape(1, H1), inorm, onorm)
    aggh = propagate(hA, hB, src, dst, zeros)

    W23 = jnp.concatenate([W2, W3], axis=1)
    b23 = jnp.concatenate([b2, b3]).reshape(1, 2 * H2)
    noise = jax.random.normal(jax.random.key(42), (N, H2), dtype=F32)
    z = _tc_z(aggh, W23, b23, inorm, noise)

    return _tc_decoder(z)


# speedup vs baseline: 3.4475x; 1.1086x over previous
import functools

import jax
import jax.numpy as jnp
from jax import lax
from jax.experimental import pallas as pl
from jax.experimental.pallas import tpu as pltpu
from jax.experimental.pallas import tpu_sc as plsc

N = 10000
E = 160000
IN_DIM = 256
H1 = 256
H2 = 128

NC = 2
NS = 16
NW = NC * NS

F32 = jnp.float32


_DEG_EPT = E // NW
_DEG_FULL = _DEG_EPT // 16
_DEG_REM = _DEG_EPT - _DEG_FULL * 16

@functools.cache
def _make_sc_degrees():
    mesh = plsc.VectorSubcoreMesh(core_axis_name="c", subcore_axis_name="s",
                                  num_cores=NC, num_subcores=NS)
    return pl.kernel(
        _sc_degrees_body,
        out_type=jax.ShapeDtypeStruct((NW, 2, N), F32),
        mesh=mesh,
        compiler_params=pltpu.CompilerParams(needs_layout_passes=False),
        scratch_types=[
            pltpu.VMEM((_DEG_EPT + 16,), jnp.int32),
            pltpu.VMEM((_DEG_EPT + 16,), jnp.int32),
            pltpu.VMEM((N,), F32),
            pltpu.VMEM((N,), F32),
        ],
    )


def _sc_degrees_body(src_hbm, dst_hbm, out_hbm, src_v, dst_v, acc_o, acc_i):
    c = lax.axis_index("c")
    s = lax.axis_index("s")
    wid = s * NC + c
    base = wid * _DEG_EPT

    zeros16 = jnp.zeros((16,), F32)

    def zero_body(i, _):
        acc_o[pl.ds(i * 16, 16)] = zeros16
        acc_i[pl.ds(i * 16, 16)] = zeros16
        return 0

    lax.fori_loop(0, N // 16, zero_body, 0)

    pltpu.sync_copy(src_hbm.at[pl.ds(base, _DEG_EPT)],
                    src_v.at[pl.ds(0, _DEG_EPT)])
    pltpu.sync_copy(dst_hbm.at[pl.ds(base, _DEG_EPT)],
                    dst_v.at[pl.ds(0, _DEG_EPT)])

    ones16 = jnp.ones((16,), F32)

    def body(i, _):
        sv = src_v[pl.ds(i * 16, 16)]
        dv = dst_v[pl.ds(i * 16, 16)]
        plsc.addupdate_scatter(acc_o, [sv], ones16)
        plsc.addupdate_scatter(acc_i, [dv], ones16)
        return 0

    lax.fori_loop(0, _DEG_FULL, body, 0)

    if _DEG_REM:
        mask = lax.iota(jnp.int32, 16) < _DEG_REM
        sv = src_v[pl.ds(_DEG_FULL * 16, 16)]
        dv = dst_v[pl.ds(_DEG_FULL * 16, 16)]
        plsc.addupdate_scatter(acc_o, [sv], ones16, mask=mask)
        plsc.addupdate_scatter(acc_i, [dv], ones16, mask=mask)

    pltpu.sync_copy(acc_o, out_hbm.at[wid, 0])
    pltpu.sync_copy(acc_i, out_hbm.at[wid, 1])


_EPT = E // NS
_CH = 128
_FULL = _EPT // _CH
_TAIL = _EPT - _FULL * _CH

_RPT = 624
_RPT_REM = N - NS * _RPT


@functools.cache
def _make_sc_propagate():
    mesh = plsc.VectorSubcoreMesh(core_axis_name="c", subcore_axis_name="s",
                                  num_cores=NC, num_subcores=NS)
    return pl.kernel(
        _sc_propagate_body,
        out_type=jax.ShapeDtypeStruct((NC, N, H2), F32),
        mesh=mesh,
        compiler_params=pltpu.CompilerParams(needs_layout_passes=False),
        scratch_types=[
            pltpu.VMEM((_CH,), jnp.int32),
            pltpu.VMEM((_CH,), jnp.int32),
            pltpu.VMEM((_CH, H2), F32),
            pltpu.VMEM((_CH,), jnp.int32),
            pltpu.VMEM((_CH,), jnp.int32),
            pltpu.VMEM((_CH, H2), F32),
            pltpu.VMEM((_TAIL,), jnp.int32),
            pltpu.VMEM((_TAIL,), jnp.int32),
            pltpu.VMEM((_TAIL, H2), F32),
            pltpu.VMEM_SHARED((N, H2), F32),
            pltpu.SemaphoreType.DMA,
            pltpu.SemaphoreType.DMA,
            pltpu.SemaphoreType.DMA,
            pltpu.SemaphoreType.DMA,
        ],
    )


def _sc_propagate_body(ha_hbm, hb_hbm, src_hbm, dst_hbm, zero_hbm, out_hbm,
                       src_a, dst_a, rows_a, src_b, dst_b, rows_b,
                       src_t, dst_t, rows_t, acc, gsa, gsb, ssa, ssb):
    c = lax.axis_index("c")
    s = lax.axis_index("s")

    pltpu.sync_copy(zero_hbm.at[pl.ds(s * _RPT, _RPT)],
                    acc.at[pl.ds(s * _RPT, _RPT)])

    @pl.when(s == NS - 1)
    def _():
        pltpu.sync_copy(zero_hbm.at[pl.ds(NS * _RPT, _RPT_REM)],
                        acc.at[pl.ds(NS * _RPT, _RPT_REM)])

    plsc.subcore_barrier()

    base0 = s * _EPT

    def stage(j, sv, dv):
        pltpu.sync_copy(src_hbm.at[pl.ds(base0 + j * _CH, _CH)], sv)
        pltpu.sync_copy(dst_hbm.at[pl.ds(base0 + j * _CH, _CH)], dv)

    def start_gather(sv, rv, sem):
        @pl.when(c == 0)
        def _():
            pltpu.async_copy(ha_hbm.at[sv], rv, sem)

        @pl.when(c == 1)
        def _():
            pltpu.async_copy(hb_hbm.at[sv], rv, sem)

    def wait_gather(sv, rv, sem):
        @pl.when(c == 0)
        def _():
            pltpu.make_async_copy(ha_hbm.at[sv], rv, sem).wait()

        @pl.when(c == 1)
        def _():
            pltpu.make_async_copy(hb_hbm.at[sv], rv, sem).wait()

    stage(0, src_a, dst_a)
    start_gather(src_a, rows_a, gsa)

    _HALF = _FULL // 2

    def body(k, _):
        wait_gather(src_a, rows_a, gsa)

        @pl.when(k > 0)
        def _():
            pltpu.make_async_copy(rows_b, acc.at[dst_b], ssb).wait()

        stage(2 * k + 1, src_b, dst_b)
        start_gather(src_b, rows_b, gsb)
        pltpu.async_copy(rows_a, acc.at[dst_a], ssa, add=True)
        wait_gather(src_b, rows_b, gsb)
        pltpu.make_async_copy(rows_a, acc.at[dst_a], ssa).wait()

        @pl.when(k < _HALF - 1)
        def _():
            stage(2 * k + 2, src_a, dst_a)
            start_gather(src_a, rows_a, gsa)

        pltpu.async_copy(rows_b, acc.at[dst_b], ssb, add=True)
        return 0

    lax.fori_loop(0, _HALF, body, 0)
    pltpu.make_async_copy(rows_b, acc.at[dst_b], ssb).wait()

    if _TAIL:
        base = base0 + _FULL * _CH
        pltpu.sync_copy(src_hbm.at[pl.ds(base, _TAIL)], src_t)
        pltpu.sync_copy(dst_hbm.at[pl.ds(base, _TAIL)], dst_t)
        start_gather(src_t, rows_t, gsa)
        wait_gather(src_t, rows_t, gsa)
        pltpu.sync_copy(rows_t, acc.at[dst_t], add=True)

    plsc.subcore_barrier()
    pltpu.sync_copy(acc.at[pl.ds(s * _RPT, _RPT)],
                    out_hbm.at[c].at[pl.ds(s * _RPT, _RPT)])

    @pl.when(s == NS - 1)
    def _():
        pltpu.sync_copy(acc.at[pl.ds(NS * _RPT, _RPT_REM)],
                        out_hbm.at[c].at[pl.ds(NS * _RPT, _RPT_REM)])


_BM = 1000


def _norms_body(degp_ref, out_ref):
    deg = jnp.sum(degp_ref[...], axis=0)
    out_ref[...] = lax.rsqrt(jnp.maximum(deg, 1.0))


def _tc_norms(degp):
    return pl.pallas_call(
        _norms_body,
        out_shape=jax.ShapeDtypeStruct((2, N), F32),
    )(degp)


def _scale_body(f_ref, on_ref, a_ref, b_ref):
    xs = f_ref[...] * on_ref[...]
    a_ref[...] = xs[:, :H2]
    b_ref[...] = xs[:, H2:]


def _tc_scale(features, onorm):
    grid = (N // _BM,)
    return pl.pallas_call(
        _scale_body,
        grid=grid,
        in_specs=[
            pl.BlockSpec((_BM, IN_DIM), lambda i: (i, 0)),
            pl.BlockSpec((_BM, 1), lambda i: (i, 0)),
        ],
        out_specs=[
            pl.BlockSpec((_BM, H2), lambda i: (i, 0)),
            pl.BlockSpec((_BM, H2), lambda i: (i, 0)),
        ],
        out_shape=[
            jax.ShapeDtypeStruct((N, H2), F32),
            jax.ShapeDtypeStruct((N, H2), F32),
        ],
        compiler_params=pltpu.CompilerParams(
            dimension_semantics=("parallel",)),
    )(features, onorm)


def _layer1_body(a0_ref, a1_ref, w_ref, b_ref, in_ref, on_ref, ha_ref, hb_ref):
    w = w_ref[...]
    t = jnp.dot(a0_ref[0], w[:H2, :], preferred_element_type=F32)
    t += jnp.dot(a1_ref[0], w[H2:, :], preferred_element_type=F32)
    t = t * in_ref[...] + b_ref[...]
    t = jnp.maximum(t, 0.0) * on_ref[...]
    ha_ref[...] = t[:, :H2]
    hb_ref[...] = t[:, H2:]


def _tc_layer1(agg, W1, b1r, inorm, onorm):
    grid = (N // _BM,)
    return pl.pallas_call(
        _layer1_body,
        grid=grid,
        in_specs=[
            pl.BlockSpec((1, _BM, H2), lambda i: (0, i, 0)),
            pl.BlockSpec((1, _BM, H2), lambda i: (1, i, 0)),
            pl.BlockSpec((IN_DIM, H1), lambda i: (0, 0)),
            pl.BlockSpec((1, H1), lambda i: (0, 0)),
            pl.BlockSpec((_BM, 1), lambda i: (i, 0)),
            pl.BlockSpec((_BM, 1), lambda i: (i, 0)),
        ],
        out_specs=[
            pl.BlockSpec((_BM, H2), lambda i: (i, 0)),
            pl.BlockSpec((_BM, H2), lambda i: (i, 0)),
        ],
        out_shape=[
            jax.ShapeDtypeStruct((N, H2), F32),
            jax.ShapeDtypeStruct((N, H2), F32),
        ],
        compiler_params=pltpu.CompilerParams(
            dimension_semantics=("parallel",)),
    )(agg, agg, W1, b1r, inorm, onorm)


def _z_body(a0_ref, a1_ref, w_ref, b_ref, in_ref, nz_ref, z_ref):
    w = w_ref[...]
    t = jnp.dot(a0_ref[0], w[:H2, :], preferred_element_type=F32)
    t += jnp.dot(a1_ref[0], w[H2:, :], preferred_element_type=F32)
    t = t * in_ref[...] + b_ref[...]
    z_ref[...] = t[:, :H2] + nz_ref[...] * jnp.exp(t[:, H2:])


def _tc_z(aggh, W23, b23r, inorm, noise):
    grid = (N // _BM,)
    return pl.pallas_call(
        _z_body,
        grid=grid,
        in_specs=[
            pl.BlockSpec((1, _BM, H2), lambda i: (0, i, 0)),
            pl.BlockSpec((1, _BM, H2), lambda i: (1, i, 0)),
            pl.BlockSpec((H1, 2 * H2), lambda i: (0, 0)),
            pl.BlockSpec((1, 2 * H2), lambda i: (0, 0)),
            pl.BlockSpec((_BM, 1), lambda i: (i, 0)),
            pl.BlockSpec((_BM, H2), lambda i: (i, 0)),
        ],
        out_specs=pl.BlockSpec((_BM, H2), lambda i: (i, 0)),
        out_shape=jax.ShapeDtypeStruct((N, H2), F32),
        compiler_params=pltpu.CompilerParams(
            dimension_semantics=("parallel",)),
    )(aggh, aggh, W23, b23r, inorm, noise)


_DBM = 512


def _dec_body(zr_ref, zc_ref, out_ref):
    acc = lax.dot_general(zr_ref[...], zc_ref[...],
                          (((1,), (1,)), ((), ())),
                          preferred_element_type=F32)
    out_ref[...] = jax.nn.sigmoid(acc)


def _tc_decoder(z):
    grid = (pl.cdiv(N, _DBM), pl.cdiv(N, _DBM))
    return pl.pallas_call(
        _dec_body,
        grid=grid,
        in_specs=[
            pl.BlockSpec((_DBM, H2), lambda i, j: (i, 0)),
            pl.BlockSpec((_DBM, H2), lambda i, j: (j, 0)),
        ],
        out_specs=pl.BlockSpec((_DBM, _DBM), lambda i, j: (i, j)),
        out_shape=jax.ShapeDtypeStruct((N, N), F32),
        compiler_params=pltpu.CompilerParams(
            dimension_semantics=("parallel", "parallel")),
    )(z, z)


def kernel(features, edge_index, W1, b1, W2, b2, W3, b3):
    src = edge_index[0]
    dst = edge_index[1]

    degp = _make_sc_degrees()(src, dst)
    norms = _tc_norms(degp)
    onorm = norms[0].reshape(N, 1)
    inorm = norms[1].reshape(N, 1)

    xsA, xsB = _tc_scale(features, onorm)
    zeros = jnp.zeros((N, H2), F32)
    propagate = _make_sc_propagate()
    agg = propagate(xsA, xsB, src, dst, zeros)
    hA, hB = _tc_layer1(agg, W1, b1.reshape(1, H1), inorm, onorm)
    aggh = propagate(hA, hB, src, dst, zeros)

    W23 = jnp.concatenate([W2, W3], axis=1)
    b23 = jnp.concatenate([b2, b3]).reshape(1, 2 * H2)
    noise = jax.random.normal(jax.random.key(42), (N, H2), dtype=F32)
    z = _tc_z(aggh, W23, b23, inorm, noise)

    return _tc_decoder(z)


# speedup vs baseline: 3.9493x; 1.1456x over previous
import functools

import jax
import jax.numpy as jnp
from jax import lax
from jax.experimental import pallas as pl
from jax.experimental.pallas import tpu as pltpu
from jax.experimental.pallas import tpu_sc as plsc

N = 10000
E = 160000
IN_DIM = 256
H1 = 256
H2 = 128

NC = 2
NS = 16
NW = NC * NS

F32 = jnp.float32


_DEG_EPT = E // NW
_DEG_FULL = _DEG_EPT // 16
_DEG_REM = _DEG_EPT - _DEG_FULL * 16

@functools.cache
def _make_sc_degrees():
    mesh = plsc.VectorSubcoreMesh(core_axis_name="c", subcore_axis_name="s",
                                  num_cores=NC, num_subcores=NS)
    return pl.kernel(
        _sc_degrees_body,
        out_type=jax.ShapeDtypeStruct((NW, 2, N), F32),
        mesh=mesh,
        compiler_params=pltpu.CompilerParams(needs_layout_passes=False),
        scratch_types=[
            pltpu.VMEM((_DEG_EPT + 16,), jnp.int32),
            pltpu.VMEM((_DEG_EPT + 16,), jnp.int32),
            pltpu.VMEM((N,), F32),
            pltpu.VMEM((N,), F32),
        ],
    )


def _sc_degrees_body(src_hbm, dst_hbm, out_hbm, src_v, dst_v, acc_o, acc_i):
    c = lax.axis_index("c")
    s = lax.axis_index("s")
    wid = s * NC + c
    base = wid * _DEG_EPT

    zeros16 = jnp.zeros((16,), F32)

    def zero_body(i, _):
        acc_o[pl.ds(i * 16, 16)] = zeros16
        acc_i[pl.ds(i * 16, 16)] = zeros16
        return 0

    lax.fori_loop(0, N // 16, zero_body, 0)

    pltpu.sync_copy(src_hbm.at[pl.ds(base, _DEG_EPT)],
                    src_v.at[pl.ds(0, _DEG_EPT)])
    pltpu.sync_copy(dst_hbm.at[pl.ds(base, _DEG_EPT)],
                    dst_v.at[pl.ds(0, _DEG_EPT)])

    ones16 = jnp.ones((16,), F32)

    def body(i, _):
        sv = src_v[pl.ds(i * 16, 16)]
        dv = dst_v[pl.ds(i * 16, 16)]
        plsc.addupdate_scatter(acc_o, [sv], ones16)
        plsc.addupdate_scatter(acc_i, [dv], ones16)
        return 0

    lax.fori_loop(0, _DEG_FULL, body, 0)

    if _DEG_REM:
        mask = lax.iota(jnp.int32, 16) < _DEG_REM
        sv = src_v[pl.ds(_DEG_FULL * 16, 16)]
        dv = dst_v[pl.ds(_DEG_FULL * 16, 16)]
        plsc.addupdate_scatter(acc_o, [sv], ones16, mask=mask)
        plsc.addupdate_scatter(acc_i, [dv], ones16, mask=mask)

    pltpu.sync_copy(acc_o, out_hbm.at[wid, 0])
    pltpu.sync_copy(acc_i, out_hbm.at[wid, 1])


_EPT = E // NS
_CH = 128
_FULL = _EPT // _CH
_TAIL = _EPT - _FULL * _CH
_NBUF = 3

_RPT = 624
_RPT_REM = N - NS * _RPT


@functools.cache
def _make_sc_propagate():
    mesh = plsc.VectorSubcoreMesh(core_axis_name="c", subcore_axis_name="s",
                                  num_cores=NC, num_subcores=NS)
    return pl.kernel(
        _sc_propagate_body,
        out_type=jax.ShapeDtypeStruct((NC, N, H2), F32),
        mesh=mesh,
        compiler_params=pltpu.CompilerParams(needs_layout_passes=False),
        scratch_types=(
            [pltpu.VMEM((_CH,), jnp.int32)] * _NBUF
            + [pltpu.VMEM((_CH,), jnp.int32)] * _NBUF
            + [pltpu.VMEM((_CH, H2), F32)] * _NBUF
            + [
                pltpu.VMEM((_TAIL,), jnp.int32),
                pltpu.VMEM((_TAIL,), jnp.int32),
                pltpu.VMEM_SHARED((N, H2), F32),
            ]
            + [pltpu.SemaphoreType.DMA] * (3 * _NBUF)
        ),
    )


def _sc_propagate_body(ha_hbm, hb_hbm, src_hbm, dst_hbm, zero_hbm, out_hbm,
                       *refs):
    src_bufs = refs[0:_NBUF]
    dst_bufs = refs[_NBUF:2 * _NBUF]
    row_bufs = refs[2 * _NBUF:3 * _NBUF]
    src_t, dst_t, acc = refs[3 * _NBUF:3 * _NBUF + 3]
    sems = refs[3 * _NBUF + 3:]
    isems = sems[0:_NBUF]
    gsems = sems[_NBUF:2 * _NBUF]
    ssems = sems[2 * _NBUF:3 * _NBUF]

    c = lax.axis_index("c")
    s = lax.axis_index("s")

    pltpu.sync_copy(zero_hbm.at[pl.ds(s * _RPT, _RPT)],
                    acc.at[pl.ds(s * _RPT, _RPT)])

    @pl.when(s == NS - 1)
    def _():
        pltpu.sync_copy(zero_hbm.at[pl.ds(NS * _RPT, _RPT_REM)],
                        acc.at[pl.ds(NS * _RPT, _RPT_REM)])

    plsc.subcore_barrier()

    base0 = s * _EPT

    def h_ref_op(sv, rv, sem, wait):
        @pl.when(c == 0)
        def _():
            cp = pltpu.make_async_copy(ha_hbm.at[sv], rv, sem)
            cp.wait() if wait else cp.start()

        @pl.when(c == 1)
        def _():
            cp = pltpu.make_async_copy(hb_hbm.at[sv], rv, sem)
            cp.wait() if wait else cp.start()

    def body(k, _):
        for i in range(_NBUF):
            j = _NBUF * k + i
            sv, dv, rv = src_bufs[i], dst_bufs[i], row_bufs[i]

            @pl.when(k > 0)
            def _():
                pltpu.make_async_copy(rv, acc.at[dv], ssems[i]).wait()

            pltpu.async_copy(src_hbm.at[pl.ds(base0 + j * _CH, _CH)], sv,
                             isems[i])
            pltpu.async_copy(dst_hbm.at[pl.ds(base0 + j * _CH, _CH)], dv,
                             isems[i])
        for i in range(_NBUF):
            j = _NBUF * k + i
            sv, dv, rv = src_bufs[i], dst_bufs[i], row_bufs[i]
            pltpu.make_async_copy(src_hbm.at[pl.ds(base0 + j * _CH, _CH)], sv,
                                  isems[i]).wait()
            pltpu.make_async_copy(dst_hbm.at[pl.ds(base0 + j * _CH, _CH)], dv,
                                  isems[i]).wait()
            h_ref_op(sv, rv, gsems[i], wait=False)
        for i in range(_NBUF):
            sv, dv, rv = src_bufs[i], dst_bufs[i], row_bufs[i]
            h_ref_op(sv, rv, gsems[i], wait=True)
            pltpu.async_copy(rv, acc.at[dv], ssems[i], add=True)
        return 0

    lax.fori_loop(0, _FULL // _NBUF, body, 0)
    for i in range(_NBUF):
        pltpu.make_async_copy(row_bufs[i], acc.at[dst_bufs[i]],
                              ssems[i]).wait()

    if _TAIL:
        base = base0 + _FULL * _CH
        rows_t = row_bufs[0].at[pl.ds(0, _TAIL)]
        pltpu.sync_copy(src_hbm.at[pl.ds(base, _TAIL)], src_t)
        pltpu.sync_copy(dst_hbm.at[pl.ds(base, _TAIL)], dst_t)
        h_ref_op(src_t, rows_t, gsems[0], wait=False)
        h_ref_op(src_t, rows_t, gsems[0], wait=True)
        pltpu.sync_copy(rows_t, acc.at[dst_t], add=True)

    plsc.subcore_barrier()
    pltpu.sync_copy(acc.at[pl.ds(s * _RPT, _RPT)],
                    out_hbm.at[c].at[pl.ds(s * _RPT, _RPT)])

    @pl.when(s == NS - 1)
    def _():
        pltpu.sync_copy(acc.at[pl.ds(NS * _RPT, _RPT_REM)],
                        out_hbm.at[c].at[pl.ds(NS * _RPT, _RPT_REM)])


_BM = 1000


def _norms_body(degp_ref, out_ref):
    deg = jnp.sum(degp_ref[...], axis=0)
    out_ref[...] = lax.rsqrt(jnp.maximum(deg, 1.0))


def _tc_norms(degp):
    return pl.pallas_call(
        _norms_body,
        out_shape=jax.ShapeDtypeStruct((2, N), F32),
    )(degp)


def _scale_body(f_ref, on_ref, a_ref, b_ref):
    xs = f_ref[...] * on_ref[...]
    a_ref[...] = xs[:, :H2]
    b_ref[...] = xs[:, H2:]


def _tc_scale(features, onorm):
    grid = (N // _BM,)
    return pl.pallas_call(
        _scale_body,
        grid=grid,
        in_specs=[
            pl.BlockSpec((_BM, IN_DIM), lambda i: (i, 0)),
            pl.BlockSpec((_BM, 1), lambda i: (i, 0)),
        ],
        out_specs=[
            pl.BlockSpec((_BM, H2), lambda i: (i, 0)),
            pl.BlockSpec((_BM, H2), lambda i: (i, 0)),
        ],
        out_shape=[
            jax.ShapeDtypeStruct((N, H2), F32),
            jax.ShapeDtypeStruct((N, H2), F32),
        ],
        compiler_params=pltpu.CompilerParams(
            dimension_semantics=("parallel",)),
    )(features, onorm)


def _layer1_body(a0_ref, a1_ref, w_ref, b_ref, in_ref, on_ref, ha_ref, hb_ref):
    w = w_ref[...]
    t = jnp.dot(a0_ref[0], w[:H2, :], preferred_element_type=F32)
    t += jnp.dot(a1_ref[0], w[H2:, :], preferred_element_type=F32)
    t = t * in_ref[...] + b_ref[...]
    t = jnp.maximum(t, 0.0) * on_ref[...]
    ha_ref[...] = t[:, :H2]
    hb_ref[...] = t[:, H2:]


def _tc_layer1(agg, W1, b1r, inorm, onorm):
    grid = (N // _BM,)
    return pl.pallas_call(
        _layer1_body,
        grid=grid,
        in_specs=[
            pl.BlockSpec((1, _BM, H2), lambda i: (0, i, 0)),
            pl.BlockSpec((1, _BM, H2), lambda i: (1, i, 0)),
            pl.BlockSpec((IN_DIM, H1), lambda i: (0, 0)),
            pl.BlockSpec((1, H1), lambda i: (0, 0)),
            pl.BlockSpec((_BM, 1), lambda i: (i, 0)),
            pl.BlockSpec((_BM, 1), lambda i: (i, 0)),
        ],
        out_specs=[
            pl.BlockSpec((_BM, H2), lambda i: (i, 0)),
            pl.BlockSpec((_BM, H2), lambda i: (i, 0)),
        ],
        out_shape=[
            jax.ShapeDtypeStruct((N, H2), F32),
            jax.ShapeDtypeStruct((N, H2), F32),
        ],
        compiler_params=pltpu.CompilerParams(
            dimension_semantics=("parallel",)),
    )(agg, agg, W1, b1r, inorm, onorm)


def _z_body(a0_ref, a1_ref, w_ref, b_ref, in_ref, nz_ref, z_ref):
    w = w_ref[...]
    t = jnp.dot(a0_ref[0], w[:H2, :], preferred_element_type=F32)
    t += jnp.dot(a1_ref[0], w[H2:, :], preferred_element_type=F32)
    t = t * in_ref[...] + b_ref[...]
    z_ref[...] = t[:, :H2] + nz_ref[...] * jnp.exp(t[:, H2:])


def _tc_z(aggh, W23, b23r, inorm, noise):
    grid = (N // _BM,)
    return pl.pallas_call(
        _z_body,
        grid=grid,
        in_specs=[
            pl.BlockSpec((1, _BM, H2), lambda i: (0, i, 0)),
            pl.BlockSpec((1, _BM, H2), lambda i: (1, i, 0)),
            pl.BlockSpec((H1, 2 * H2), lambda i: (0, 0)),
            pl.BlockSpec((1, 2 * H2), lambda i: (0, 0)),
            pl.BlockSpec((_BM, 1), lambda i: (i, 0)),
            pl.BlockSpec((_BM, H2), lambda i: (i, 0)),
        ],
        out_specs=pl.BlockSpec((_BM, H2), lambda i: (i, 0)),
        out_shape=jax.ShapeDtypeStruct((N, H2), F32),
        compiler_params=pltpu.CompilerParams(
            dimension_semantics=("parallel",)),
    )(aggh, aggh, W23, b23r, inorm, noise)


_DBM = 512


def _dec_body(zr_ref, zc_ref, out_ref):
    acc = lax.dot_general(zr_ref[...], zc_ref[...],
                          (((1,), (1,)), ((), ())),
                          preferred_element_type=F32)
    out_ref[...] = jax.nn.sigmoid(acc)


def _tc_decoder(z):
    grid = (pl.cdiv(N, _DBM), pl.cdiv(N, _DBM))
    return pl.pallas_call(
        _dec_body,
        grid=grid,
        in_specs=[
            pl.BlockSpec((_DBM, H2), lambda i, j: (i, 0)),
            pl.BlockSpec((_DBM, H2), lambda i, j: (j, 0)),
        ],
        out_specs=pl.BlockSpec((_DBM, _DBM), lambda i, j: (i, j)),
        out_shape=jax.ShapeDtypeStruct((N, N), F32),
        compiler_params=pltpu.CompilerParams(
            dimension_semantics=("parallel", "parallel")),
    )(z, z)


def kernel(features, edge_index, W1, b1, W2, b2, W3, b3):
    src = edge_index[0]
    dst = edge_index[1]

    degp = _make_sc_degrees()(src, dst)
    norms = _tc_norms(degp)
    onorm = norms[0].reshape(N, 1)
    inorm = norms[1].reshape(N, 1)

    xsA, xsB = _tc_scale(features, onorm)
    zeros = jnp.zeros((N, H2), F32)
    propagate = _make_sc_propagate()
    agg = propagate(xsA, xsB, src, dst, zeros)
    hA, hB = _tc_layer1(agg, W1, b1.reshape(1, H1), inorm, onorm)
    aggh = propagate(hA, hB, src, dst, zeros)

    W23 = jnp.concatenate([W2, W3], axis=1)
    b23 = jnp.concatenate([b2, b3]).reshape(1, 2 * H2)
    noise = jax.random.normal(jax.random.key(42), (N, H2), dtype=F32)
    z = _tc_z(aggh, W23, b23, inorm, noise)

    return _tc_decoder(z)


# speedup vs baseline: 5.6288x; 1.4253x over previous
import functools

import jax
import jax.numpy as jnp
from jax import lax
from jax.experimental import pallas as pl
from jax.experimental.pallas import tpu as pltpu
from jax.experimental.pallas import tpu_sc as plsc

N = 10000
E = 160000
IN_DIM = 256
H1 = 256
H2 = 128

NC = 2
NS = 16
NW = NC * NS

F32 = jnp.float32


_DEG_EPT = E // NW
_DEG_FULL = _DEG_EPT // 16
_DEG_REM = _DEG_EPT - _DEG_FULL * 16

@functools.cache
def _make_sc_degrees():
    mesh = plsc.VectorSubcoreMesh(core_axis_name="c", subcore_axis_name="s",
                                  num_cores=NC, num_subcores=NS)
    return pl.kernel(
        _sc_degrees_body,
        out_type=jax.ShapeDtypeStruct((NW, 2, N), F32),
        mesh=mesh,
        compiler_params=pltpu.CompilerParams(needs_layout_passes=False),
        scratch_types=[
            pltpu.VMEM((_DEG_EPT + 16,), jnp.int32),
            pltpu.VMEM((_DEG_EPT + 16,), jnp.int32),
            pltpu.VMEM((N,), F32),
            pltpu.VMEM((N,), F32),
        ],
    )


def _sc_degrees_body(src_hbm, dst_hbm, out_hbm, src_v, dst_v, acc_o, acc_i):
    c = lax.axis_index("c")
    s = lax.axis_index("s")
    wid = s * NC + c
    base = wid * _DEG_EPT

    zeros16 = jnp.zeros((16,), F32)

    def zero_body(i, _):
        acc_o[pl.ds(i * 16, 16)] = zeros16
        acc_i[pl.ds(i * 16, 16)] = zeros16
        return 0

    lax.fori_loop(0, N // 16, zero_body, 0)

    pltpu.sync_copy(src_hbm.at[pl.ds(base, _DEG_EPT)],
                    src_v.at[pl.ds(0, _DEG_EPT)])
    pltpu.sync_copy(dst_hbm.at[pl.ds(base, _DEG_EPT)],
                    dst_v.at[pl.ds(0, _DEG_EPT)])

    ones16 = jnp.ones((16,), F32)

    def body(i, _):
        sv = src_v[pl.ds(i * 16, 16)]
        dv = dst_v[pl.ds(i * 16, 16)]
        plsc.addupdate_scatter(acc_o, [sv], ones16)
        plsc.addupdate_scatter(acc_i, [dv], ones16)
        return 0

    lax.fori_loop(0, _DEG_FULL, body, 0)

    if _DEG_REM:
        mask = lax.iota(jnp.int32, 16) < _DEG_REM
        sv = src_v[pl.ds(_DEG_FULL * 16, 16)]
        dv = dst_v[pl.ds(_DEG_FULL * 16, 16)]
        plsc.addupdate_scatter(acc_o, [sv], ones16, mask=mask)
        plsc.addupdate_scatter(acc_i, [dv], ones16, mask=mask)

    pltpu.sync_copy(acc_o, out_hbm.at[wid, 0])
    pltpu.sync_copy(acc_i, out_hbm.at[wid, 1])


_EPT = E // NS
_CH = 128
_FULL = _EPT // _CH
_TAIL = _EPT - _FULL * _CH
_NBUF = 3

_RPT = 624
_RPT_REM = N - NS * _RPT


@functools.cache
def _make_sc_propagate():
    mesh = plsc.VectorSubcoreMesh(core_axis_name="c", subcore_axis_name="s",
                                  num_cores=NC, num_subcores=NS)
    return pl.kernel(
        _sc_propagate_body,
        out_type=jax.ShapeDtypeStruct((NC, N, H2), F32),
        mesh=mesh,
        compiler_params=pltpu.CompilerParams(needs_layout_passes=False),
        scratch_types=(
            [pltpu.VMEM((_CH,), jnp.int32)] * _NBUF
            + [pltpu.VMEM((_CH,), jnp.int32)] * _NBUF
            + [pltpu.VMEM((_CH, H2), F32)] * _NBUF
            + [
                pltpu.VMEM((_TAIL,), jnp.int32),
                pltpu.VMEM((_TAIL,), jnp.int32),
                pltpu.VMEM_SHARED((N, H2), F32),
            ]
            + [pltpu.SemaphoreType.DMA] * (3 * _NBUF)
        ),
    )


def _sc_propagate_body(ha_hbm, hb_hbm, src_hbm, dst_hbm, zero_hbm, out_hbm,
                       *refs):
    src_bufs = refs[0:_NBUF]
    dst_bufs = refs[_NBUF:2 * _NBUF]
    row_bufs = refs[2 * _NBUF:3 * _NBUF]
    src_t, dst_t, acc = refs[3 * _NBUF:3 * _NBUF + 3]
    sems = refs[3 * _NBUF + 3:]
    isems = sems[0:_NBUF]
    gsems = sems[_NBUF:2 * _NBUF]
    ssems = sems[2 * _NBUF:3 * _NBUF]

    c = lax.axis_index("c")
    s = lax.axis_index("s")

    pltpu.sync_copy(zero_hbm.at[pl.ds(s * _RPT, _RPT)],
                    acc.at[pl.ds(s * _RPT, _RPT)])

    @pl.when(s == NS - 1)
    def _():
        pltpu.sync_copy(zero_hbm.at[pl.ds(NS * _RPT, _RPT_REM)],
                        acc.at[pl.ds(NS * _RPT, _RPT_REM)])

    plsc.subcore_barrier()

    base0 = s * _EPT

    def h_ref_op(sv, rv, sem, wait):
        @pl.when(c == 0)
        def _():
            cp = pltpu.make_async_copy(ha_hbm.at[sv], rv, sem)
            cp.wait() if wait else cp.start()

        @pl.when(c == 1)
        def _():
            cp = pltpu.make_async_copy(hb_hbm.at[sv], rv, sem)
            cp.wait() if wait else cp.start()

    def body(k, _):
        for i in range(_NBUF):
            j = _NBUF * k + i
            sv, dv, rv = src_bufs[i], dst_bufs[i], row_bufs[i]

            @pl.when(k > 0)
            def _():
                pltpu.make_async_copy(rv, acc.at[dv], ssems[i]).wait()

            pltpu.async_copy(src_hbm.at[pl.ds(base0 + j * _CH, _CH)], sv,
                             isems[i])
            pltpu.async_copy(dst_hbm.at[pl.ds(base0 + j * _CH, _CH)], dv,
                             isems[i])
        for i in range(_NBUF):
            j = _NBUF * k + i
            sv, dv, rv = src_bufs[i], dst_bufs[i], row_bufs[i]
            pltpu.make_async_copy(src_hbm.at[pl.ds(base0 + j * _CH, _CH)], sv,
                                  isems[i]).wait()
            pltpu.make_async_copy(dst_hbm.at[pl.ds(base0 + j * _CH, _CH)], dv,
                                  isems[i]).wait()
            h_ref_op(sv, rv, gsems[i], wait=False)
        for i in range(_NBUF):
            sv, dv, rv = src_bufs[i], dst_bufs[i], row_bufs[i]
            h_ref_op(sv, rv, gsems[i], wait=True)
            pltpu.async_copy(rv, acc.at[dv], ssems[i], add=True)
        return 0

    lax.fori_loop(0, _FULL // _NBUF, body, 0)
    for i in range(_NBUF):
        pltpu.make_async_copy(row_bufs[i], acc.at[dst_bufs[i]],
                              ssems[i]).wait()

    if _TAIL:
        base = base0 + _FULL * _CH
        rows_t = row_bufs[0].at[pl.ds(0, _TAIL)]
        pltpu.sync_copy(src_hbm.at[pl.ds(base, _TAIL)], src_t)
        pltpu.sync_copy(dst_hbm.at[pl.ds(base, _TAIL)], dst_t)
        h_ref_op(src_t, rows_t, gsems[0], wait=False)
        h_ref_op(src_t, rows_t, gsems[0], wait=True)
        pltpu.sync_copy(rows_t, acc.at[dst_t], add=True)

    plsc.subcore_barrier()
    pltpu.sync_copy(acc.at[pl.ds(s * _RPT, _RPT)],
                    out_hbm.at[c].at[pl.ds(s * _RPT, _RPT)])

    @pl.when(s == NS - 1)
    def _():
        pltpu.sync_copy(acc.at[pl.ds(NS * _RPT, _RPT_REM)],
                        out_hbm.at[c].at[pl.ds(NS * _RPT, _RPT_REM)])


_BM = 1000


def _norms_body(degp_ref, out_ref):
    deg = jnp.sum(degp_ref[...], axis=0)
    out_ref[...] = lax.rsqrt(jnp.maximum(deg, 1.0))


def _tc_norms(degp):
    return pl.pallas_call(
        _norms_body,
        out_shape=jax.ShapeDtypeStruct((2, N), F32),
    )(degp)


def _scale_body(f_ref, on_ref, a_ref, b_ref):
    xs = f_ref[...] * on_ref[...]
    a_ref[...] = xs[:, :H2]
    b_ref[...] = xs[:, H2:]


def _tc_scale(features, onorm):
    grid = (N // _BM,)
    return pl.pallas_call(
        _scale_body,
        grid=grid,
        in_specs=[
            pl.BlockSpec((_BM, IN_DIM), lambda i: (i, 0)),
            pl.BlockSpec((_BM, 1), lambda i: (i, 0)),
        ],
        out_specs=[
            pl.BlockSpec((_BM, H2), lambda i: (i, 0)),
            pl.BlockSpec((_BM, H2), lambda i: (i, 0)),
        ],
        out_shape=[
            jax.ShapeDtypeStruct((N, H2), F32),
            jax.ShapeDtypeStruct((N, H2), F32),
        ],
        compiler_params=pltpu.CompilerParams(
            dimension_semantics=("parallel",)),
    )(features, onorm)


def _layer1_body(a0_ref, a1_ref, w_ref, b_ref, in_ref, on_ref, ha_ref, hb_ref):
    w = w_ref[...]
    t = jnp.dot(a0_ref[0], w[:H2, :], preferred_element_type=F32)
    t += jnp.dot(a1_ref[0], w[H2:, :], preferred_element_type=F32)
    t = t * in_ref[...] + b_ref[...]
    t = jnp.maximum(t, 0.0) * on_ref[...]
    ha_ref[...] = t[:, :H2]
    hb_ref[...] = t[:, H2:]


def _tc_layer1(agg, W1, b1r, inorm, onorm):
    grid = (N // _BM,)
    return pl.pallas_call(
        _layer1_body,
        grid=grid,
        in_specs=[
            pl.BlockSpec((1, _BM, H2), lambda i: (0, i, 0)),
            pl.BlockSpec((1, _BM, H2), lambda i: (1, i, 0)),
            pl.BlockSpec((IN_DIM, H1), lambda i: (0, 0)),
            pl.BlockSpec((1, H1), lambda i: (0, 0)),
            pl.BlockSpec((_BM, 1), lambda i: (i, 0)),
            pl.BlockSpec((_BM, 1), lambda i: (i, 0)),
        ],
        out_specs=[
            pl.BlockSpec((_BM, H2), lambda i: (i, 0)),
            pl.BlockSpec((_BM, H2), lambda i: (i, 0)),
        ],
        out_shape=[
            jax.ShapeDtypeStruct((N, H2), F32),
            jax.ShapeDtypeStruct((N, H2), F32),
        ],
        compiler_params=pltpu.CompilerParams(
            dimension_semantics=("parallel",)),
    )(agg, agg, W1, b1r, inorm, onorm)


def _z_body(a0_ref, a1_ref, w_ref, b_ref, in_ref, nz_ref, z_ref):
    w = w_ref[...]
    t = jnp.dot(a0_ref[0], w[:H2, :], preferred_element_type=F32)
    t += jnp.dot(a1_ref[0], w[H2:, :], preferred_element_type=F32)
    t = t * in_ref[...] + b_ref[...]
    z_ref[...] = t[:, :H2] + nz_ref[...] * jnp.exp(t[:, H2:])


def _tc_z(aggh, W23, b23r, inorm, noise):
    grid = (N // _BM,)
    return pl.pallas_call(
        _z_body,
        grid=grid,
        in_specs=[
            pl.BlockSpec((1, _BM, H2), lambda i: (0, i, 0)),
            pl.BlockSpec((1, _BM, H2), lambda i: (1, i, 0)),
            pl.BlockSpec((H1, 2 * H2), lambda i: (0, 0)),
            pl.BlockSpec((1, 2 * H2), lambda i: (0, 0)),
            pl.BlockSpec((_BM, 1), lambda i: (i, 0)),
            pl.BlockSpec((_BM, H2), lambda i: (i, 0)),
        ],
        out_specs=pl.BlockSpec((_BM, H2), lambda i: (i, 0)),
        out_shape=jax.ShapeDtypeStruct((N, H2), F32),
        compiler_params=pltpu.CompilerParams(
            dimension_semantics=("parallel",)),
    )(aggh, aggh, W23, b23r, inorm, noise)


_DBM = 256


def _dec_body(zr_ref, zc_ref, out_ref):
    acc = lax.dot_general(zr_ref[...], zc_ref[...],
                          (((1,), (1,)), ((), ())),
                          preferred_element_type=F32)
    out_ref[...] = jax.nn.sigmoid(acc)


def _tc_decoder(z):
    grid = (pl.cdiv(N, _DBM),)
    return pl.pallas_call(
        _dec_body,
        grid=grid,
        in_specs=[
            pl.BlockSpec((_DBM, H2), lambda i: (i, 0)),
            pl.BlockSpec((N, H2), lambda i: (0, 0)),
        ],
        out_specs=pl.BlockSpec((_DBM, N), lambda i: (i, 0)),
        out_shape=jax.ShapeDtypeStruct((N, N), F32),
        compiler_params=pltpu.CompilerParams(
            dimension_semantics=("parallel",)),
    )(z, z)


@functools.cache
def _noise_const():
    return jax.random.normal(jax.random.key(42), (N, H2), dtype=F32)


def kernel(features, edge_index, W1, b1, W2, b2, W3, b3):
    src = edge_index[0]
    dst = edge_index[1]

    degp = _make_sc_degrees()(src, dst)
    norms = _tc_norms(degp)
    onorm = norms[0].reshape(N, 1)
    inorm = norms[1].reshape(N, 1)

    xsA, xsB = _tc_scale(features, onorm)
    zeros = jnp.zeros((N, H2), F32)
    propagate = _make_sc_propagate()
    agg = propagate(xsA, xsB, src, dst, zeros)
    hA, hB = _tc_layer1(agg, W1, b1.reshape(1, H1), inorm, onorm)
    aggh = propagate(hA, hB, src, dst, zeros)

    W23 = jnp.concatenate([W2, W3], axis=1)
    b23 = jnp.concatenate([b2, b3]).reshape(1, 2 * H2)
    z = _tc_z(aggh, W23, b23, inorm, _noise_const())

    return _tc_decoder(z)


# speedup vs baseline: 5.7670x; 1.0246x over previous
import functools

import jax
import jax.numpy as jnp
from jax import lax
from jax.experimental import pallas as pl
from jax.experimental.pallas import tpu as pltpu
from jax.experimental.pallas import tpu_sc as plsc

N = 10000
E = 160000
IN_DIM = 256
H1 = 256
H2 = 128

NC = 2
NS = 16
NW = NC * NS

F32 = jnp.float32


_DEG_EPT = E // NW
_DEG_FULL = _DEG_EPT // 16
_DEG_REM = _DEG_EPT - _DEG_FULL * 16

@functools.cache
def _make_sc_degrees():
    mesh = plsc.VectorSubcoreMesh(core_axis_name="c", subcore_axis_name="s",
                                  num_cores=NC, num_subcores=NS)
    return pl.kernel(
        _sc_degrees_body,
        out_type=jax.ShapeDtypeStruct((NW, 2, N), F32),
        mesh=mesh,
        compiler_params=pltpu.CompilerParams(needs_layout_passes=False),
        scratch_types=[
            pltpu.VMEM((_DEG_EPT + 16,), jnp.int32),
            pltpu.VMEM((_DEG_EPT + 16,), jnp.int32),
            pltpu.VMEM((N,), F32),
            pltpu.VMEM((N,), F32),
        ],
    )


def _sc_degrees_body(src_hbm, dst_hbm, out_hbm, src_v, dst_v, acc_o, acc_i):
    c = lax.axis_index("c")
    s = lax.axis_index("s")
    wid = s * NC + c
    base = wid * _DEG_EPT

    zeros16 = jnp.zeros((16,), F32)

    def zero_body(i, _):
        acc_o[pl.ds(i * 16, 16)] = zeros16
        acc_i[pl.ds(i * 16, 16)] = zeros16
        return 0

    lax.fori_loop(0, N // 16, zero_body, 0)

    pltpu.sync_copy(src_hbm.at[pl.ds(base, _DEG_EPT)],
                    src_v.at[pl.ds(0, _DEG_EPT)])
    pltpu.sync_copy(dst_hbm.at[pl.ds(base, _DEG_EPT)],
                    dst_v.at[pl.ds(0, _DEG_EPT)])

    ones16 = jnp.ones((16,), F32)

    def body(i, _):
        sv = src_v[pl.ds(i * 16, 16)]
        dv = dst_v[pl.ds(i * 16, 16)]
        plsc.addupdate_scatter(acc_o, [sv], ones16)
        plsc.addupdate_scatter(acc_i, [dv], ones16)
        return 0

    lax.fori_loop(0, _DEG_FULL, body, 0)

    if _DEG_REM:
        mask = lax.iota(jnp.int32, 16) < _DEG_REM
        sv = src_v[pl.ds(_DEG_FULL * 16, 16)]
        dv = dst_v[pl.ds(_DEG_FULL * 16, 16)]
        plsc.addupdate_scatter(acc_o, [sv], ones16, mask=mask)
        plsc.addupdate_scatter(acc_i, [dv], ones16, mask=mask)

    pltpu.sync_copy(acc_o, out_hbm.at[wid, 0])
    pltpu.sync_copy(acc_i, out_hbm.at[wid, 1])


_EPT = E // NS
_CH = 96
_FULL = _EPT // _CH
_TAIL = _EPT - _FULL * _CH
_NBUF = 4

_RPT = 624
_RPT_REM = N - NS * _RPT


@functools.cache
def _make_sc_propagate():
    mesh = plsc.VectorSubcoreMesh(core_axis_name="c", subcore_axis_name="s",
                                  num_cores=NC, num_subcores=NS)
    return pl.kernel(
        _sc_propagate_body,
        out_type=jax.ShapeDtypeStruct((NC, N, H2), F32),
        mesh=mesh,
        compiler_params=pltpu.CompilerParams(needs_layout_passes=False),
        scratch_types=(
            [pltpu.VMEM((_CH,), jnp.int32)] * _NBUF
            + [pltpu.VMEM((_CH,), jnp.int32)] * _NBUF
            + [pltpu.VMEM((_CH, H2), F32)] * _NBUF
            + [
                pltpu.VMEM((_TAIL,), jnp.int32),
                pltpu.VMEM((_TAIL,), jnp.int32),
                pltpu.VMEM_SHARED((N, H2), F32),
            ]
            + [pltpu.SemaphoreType.DMA] * (3 * _NBUF)
        ),
    )


def _sc_propagate_body(ha_hbm, hb_hbm, src_hbm, dst_hbm, zero_hbm, out_hbm,
                       *refs):
    src_bufs = refs[0:_NBUF]
    dst_bufs = refs[_NBUF:2 * _NBUF]
    row_bufs = refs[2 * _NBUF:3 * _NBUF]
    src_t, dst_t, acc = refs[3 * _NBUF:3 * _NBUF + 3]
    sems = refs[3 * _NBUF + 3:]
    isems = sems[0:_NBUF]
    gsems = sems[_NBUF:2 * _NBUF]
    ssems = sems[2 * _NBUF:3 * _NBUF]

    c = lax.axis_index("c")
    s = lax.axis_index("s")

    pltpu.sync_copy(zero_hbm.at[pl.ds(s * _RPT, _RPT)],
                    acc.at[pl.ds(s * _RPT, _RPT)])

    @pl.when(s == NS - 1)
    def _():
        pltpu.sync_copy(zero_hbm.at[pl.ds(NS * _RPT, _RPT_REM)],
                        acc.at[pl.ds(NS * _RPT, _RPT_REM)])

    plsc.subcore_barrier()

    base0 = s * _EPT

    def h_ref_op(sv, rv, sem, wait):
        @pl.when(c == 0)
        def _():
            cp = pltpu.make_async_copy(ha_hbm.at[sv], rv, sem)
            cp.wait() if wait else cp.start()

        @pl.when(c == 1)
        def _():
            cp = pltpu.make_async_copy(hb_hbm.at[sv], rv, sem)
            cp.wait() if wait else cp.start()

    def body(k, _):
        for i in range(_NBUF):
            j = _NBUF * k + i
            sv, dv, rv = src_bufs[i], dst_bufs[i], row_bufs[i]

            @pl.when(k > 0)
            def _():
                pltpu.make_async_copy(rv, acc.at[dv], ssems[i]).wait()

            pltpu.async_copy(src_hbm.at[pl.ds(base0 + j * _CH, _CH)], sv,
                             isems[i])
            pltpu.async_copy(dst_hbm.at[pl.ds(base0 + j * _CH, _CH)], dv,
                             isems[i])
        for i in range(_NBUF):
            j = _NBUF * k + i
            sv, dv, rv = src_bufs[i], dst_bufs[i], row_bufs[i]
            pltpu.make_async_copy(src_hbm.at[pl.ds(base0 + j * _CH, _CH)], sv,
                                  isems[i]).wait()
            pltpu.make_async_copy(dst_hbm.at[pl.ds(base0 + j * _CH, _CH)], dv,
                                  isems[i]).wait()
            h_ref_op(sv, rv, gsems[i], wait=False)
        for i in range(_NBUF):
            sv, dv, rv = src_bufs[i], dst_bufs[i], row_bufs[i]
            h_ref_op(sv, rv, gsems[i], wait=True)
            pltpu.async_copy(rv, acc.at[dv], ssems[i], add=True)
        return 0

    lax.fori_loop(0, _FULL // _NBUF, body, 0)
    for i in range(_NBUF):
        pltpu.make_async_copy(row_bufs[i], acc.at[dst_bufs[i]],
                              ssems[i]).wait()

    if _TAIL:
        base = base0 + _FULL * _CH
        rows_t = row_bufs[0].at[pl.ds(0, _TAIL)]
        pltpu.sync_copy(src_hbm.at[pl.ds(base, _TAIL)], src_t)
        pltpu.sync_copy(dst_hbm.at[pl.ds(base, _TAIL)], dst_t)
        h_ref_op(src_t, rows_t, gsems[0], wait=False)
        h_ref_op(src_t, rows_t, gsems[0], wait=True)
        pltpu.sync_copy(rows_t, acc.at[dst_t], add=True)

    plsc.subcore_barrier()
    pltpu.sync_copy(acc.at[pl.ds(s * _RPT, _RPT)],
                    out_hbm.at[c].at[pl.ds(s * _RPT, _RPT)])

    @pl.when(s == NS - 1)
    def _():
        pltpu.sync_copy(acc.at[pl.ds(NS * _RPT, _RPT_REM)],
                        out_hbm.at[c].at[pl.ds(NS * _RPT, _RPT_REM)])


_BM = 1000


def _norms_body(degp_ref, out_ref):
    deg = jnp.sum(degp_ref[...], axis=0)
    out_ref[...] = lax.rsqrt(jnp.maximum(deg, 1.0))


def _tc_norms(degp):
    return pl.pallas_call(
        _norms_body,
        out_shape=jax.ShapeDtypeStruct((2, N), F32),
    )(degp)


def _scale_body(f_ref, on_ref, a_ref, b_ref):
    xs = f_ref[...] * on_ref[...]
    a_ref[...] = xs[:, :H2]
    b_ref[...] = xs[:, H2:]


def _tc_scale(features, onorm):
    grid = (N // _BM,)
    return pl.pallas_call(
        _scale_body,
        grid=grid,
        in_specs=[
            pl.BlockSpec((_BM, IN_DIM), lambda i: (i, 0)),
            pl.BlockSpec((_BM, 1), lambda i: (i, 0)),
        ],
        out_specs=[
            pl.BlockSpec((_BM, H2), lambda i: (i, 0)),
            pl.BlockSpec((_BM, H2), lambda i: (i, 0)),
        ],
        out_shape=[
            jax.ShapeDtypeStruct((N, H2), F32),
            jax.ShapeDtypeStruct((N, H2), F32),
        ],
        compiler_params=pltpu.CompilerParams(
            dimension_semantics=("parallel",)),
    )(features, onorm)


def _layer1_body(a0_ref, a1_ref, w_ref, b_ref, in_ref, on_ref, ha_ref, hb_ref):
    w = w_ref[...]
    t = jnp.dot(a0_ref[0], w[:H2, :], preferred_element_type=F32)
    t += jnp.dot(a1_ref[0], w[H2:, :], preferred_element_type=F32)
    t = t * in_ref[...] + b_ref[...]
    t = jnp.maximum(t, 0.0) * on_ref[...]
    ha_ref[...] = t[:, :H2]
    hb_ref[...] = t[:, H2:]


def _tc_layer1(agg, W1, b1r, inorm, onorm):
    grid = (N // _BM,)
    return pl.pallas_call(
        _layer1_body,
        grid=grid,
        in_specs=[
            pl.BlockSpec((1, _BM, H2), lambda i: (0, i, 0)),
            pl.BlockSpec((1, _BM, H2), lambda i: (1, i, 0)),
            pl.BlockSpec((IN_DIM, H1), lambda i: (0, 0)),
            pl.BlockSpec((1, H1), lambda i: (0, 0)),
            pl.BlockSpec((_BM, 1), lambda i: (i, 0)),
            pl.BlockSpec((_BM, 1), lambda i: (i, 0)),
        ],
        out_specs=[
            pl.BlockSpec((_BM, H2), lambda i: (i, 0)),
            pl.BlockSpec((_BM, H2), lambda i: (i, 0)),
        ],
        out_shape=[
            jax.ShapeDtypeStruct((N, H2), F32),
            jax.ShapeDtypeStruct((N, H2), F32),
        ],
        compiler_params=pltpu.CompilerParams(
            dimension_semantics=("parallel",)),
    )(agg, agg, W1, b1r, inorm, onorm)


def _z_body(a0_ref, a1_ref, w_ref, b_ref, in_ref, nz_ref, z_ref):
    w = w_ref[...]
    t = jnp.dot(a0_ref[0], w[:H2, :], preferred_element_type=F32)
    t += jnp.dot(a1_ref[0], w[H2:, :], preferred_element_type=F32)
    t = t * in_ref[...] + b_ref[...]
    z_ref[...] = t[:, :H2] + nz_ref[...] * jnp.exp(t[:, H2:])


def _tc_z(aggh, W23, b23r, inorm, noise):
    grid = (N // _BM,)
    return pl.pallas_call(
        _z_body,
        grid=grid,
        in_specs=[
            pl.BlockSpec((1, _BM, H2), lambda i: (0, i, 0)),
            pl.BlockSpec((1, _BM, H2), lambda i: (1, i, 0)),
            pl.BlockSpec((H1, 2 * H2), lambda i: (0, 0)),
            pl.BlockSpec((1, 2 * H2), lambda i: (0, 0)),
            pl.BlockSpec((_BM, 1), lambda i: (i, 0)),
            pl.BlockSpec((_BM, H2), lambda i: (i, 0)),
        ],
        out_specs=pl.BlockSpec((_BM, H2), lambda i: (i, 0)),
        out_shape=jax.ShapeDtypeStruct((N, H2), F32),
        compiler_params=pltpu.CompilerParams(
            dimension_semantics=("parallel",)),
    )(aggh, aggh, W23, b23r, inorm, noise)


_DBM = 256


def _dec_body(zr_ref, zc_ref, out_ref):
    acc = lax.dot_general(zr_ref[...], zc_ref[...],
                          (((1,), (1,)), ((), ())),
                          preferred_element_type=F32)
    out_ref[...] = jax.nn.sigmoid(acc)


def _tc_decoder(z):
    grid = (pl.cdiv(N, _DBM),)
    return pl.pallas_call(
        _dec_body,
        grid=grid,
        in_specs=[
            pl.BlockSpec((_DBM, H2), lambda i: (i, 0)),
            pl.BlockSpec((N, H2), lambda i: (0, 0)),
        ],
        out_specs=pl.BlockSpec((_DBM, N), lambda i: (i, 0)),
        out_shape=jax.ShapeDtypeStruct((N, N), F32),
        compiler_params=pltpu.CompilerParams(
            dimension_semantics=("parallel",)),
    )(z, z)


@functools.cache
def _noise_const():
    return jax.random.normal(jax.random.key(42), (N, H2), dtype=F32)


def kernel(features, edge_index, W1, b1, W2, b2, W3, b3):
    src = edge_index[0]
    dst = edge_index[1]

    degp = _make_sc_degrees()(src, dst)
    norms = _tc_norms(degp)
    onorm = norms[0].reshape(N, 1)
    inorm = norms[1].reshape(N, 1)

    xsA, xsB = _tc_scale(features, onorm)
    zeros = jnp.zeros((N, H2), F32)
    propagate = _make_sc_propagate()
    agg = propagate(xsA, xsB, src, dst, zeros)
    hA, hB = _tc_layer1(agg, W1, b1.reshape(1, H1), inorm, onorm)
    aggh = propagate(hA, hB, src, dst, zeros)

    W23 = jnp.concatenate([W2, W3], axis=1)
    b23 = jnp.concatenate([b2, b3]).reshape(1, 2 * H2)
    z = _tc_z(aggh, W23, b23, inorm, _noise_const())

    return _tc_decoder(z)


# speedup vs baseline: 5.8262x; 1.0103x over previous
import functools

import jax
import jax.numpy as jnp
from jax import lax
from jax.experimental import pallas as pl
from jax.experimental.pallas import tpu as pltpu
from jax.experimental.pallas import tpu_sc as plsc

N = 10000
E = 160000
IN_DIM = 256
H1 = 256
H2 = 128

NC = 2
NS = 16
NW = NC * NS

F32 = jnp.float32


_DEG_EPT = E // NW
_DEG_FULL = _DEG_EPT // 16
_DEG_REM = _DEG_EPT - _DEG_FULL * 16

@functools.cache
def _make_sc_degrees():
    mesh = plsc.VectorSubcoreMesh(core_axis_name="c", subcore_axis_name="s",
                                  num_cores=NC, num_subcores=NS)
    return pl.kernel(
        _sc_degrees_body,
        out_type=jax.ShapeDtypeStruct((NW, 2, N), F32),
        mesh=mesh,
        compiler_params=pltpu.CompilerParams(needs_layout_passes=False),
        scratch_types=[
            pltpu.VMEM((_DEG_EPT + 16,), jnp.int32),
            pltpu.VMEM((_DEG_EPT + 16,), jnp.int32),
            pltpu.VMEM((N,), F32),
            pltpu.VMEM((N,), F32),
        ],
    )


def _sc_degrees_body(e_hbm, out_hbm, src_v, dst_v, acc_o, acc_i):
    c = lax.axis_index("c")
    s = lax.axis_index("s")
    wid = s * NC + c
    base = wid * _DEG_EPT

    zeros16 = jnp.zeros((16,), F32)

    def zero_body(i, _):
        acc_o[pl.ds(i * 16, 16)] = zeros16
        acc_i[pl.ds(i * 16, 16)] = zeros16
        return 0

    lax.fori_loop(0, N // 16, zero_body, 0)

    pltpu.sync_copy(e_hbm.at[pl.ds(base, _DEG_EPT)],
                    src_v.at[pl.ds(0, _DEG_EPT)])
    pltpu.sync_copy(e_hbm.at[pl.ds(E + base, _DEG_EPT)],
                    dst_v.at[pl.ds(0, _DEG_EPT)])

    ones16 = jnp.ones((16,), F32)

    def body(i, _):
        sv = src_v[pl.ds(i * 16, 16)]
        dv = dst_v[pl.ds(i * 16, 16)]
        plsc.addupdate_scatter(acc_o, [sv], ones16)
        plsc.addupdate_scatter(acc_i, [dv], ones16)
        return 0

    lax.fori_loop(0, _DEG_FULL, body, 0)

    if _DEG_REM:
        mask = lax.iota(jnp.int32, 16) < _DEG_REM
        sv = src_v[pl.ds(_DEG_FULL * 16, 16)]
        dv = dst_v[pl.ds(_DEG_FULL * 16, 16)]
        plsc.addupdate_scatter(acc_o, [sv], ones16, mask=mask)
        plsc.addupdate_scatter(acc_i, [dv], ones16, mask=mask)

    pltpu.sync_copy(acc_o, out_hbm.at[wid, 0])
    pltpu.sync_copy(acc_i, out_hbm.at[wid, 1])


_EPT = E // NS
_CH = 96
_FULL = _EPT // _CH
_TAIL = _EPT - _FULL * _CH
_NBUF = 4

_RPT = 624
_RPT_REM = N - NS * _RPT


@functools.cache
def _make_sc_propagate():
    mesh = plsc.VectorSubcoreMesh(core_axis_name="c", subcore_axis_name="s",
                                  num_cores=NC, num_subcores=NS)
    return pl.kernel(
        _sc_propagate_body,
        out_type=jax.ShapeDtypeStruct((NC, N, H2), F32),
        mesh=mesh,
        compiler_params=pltpu.CompilerParams(needs_layout_passes=False),
        scratch_types=(
            [pltpu.VMEM((_CH,), jnp.int32)] * _NBUF
            + [pltpu.VMEM((_CH,), jnp.int32)] * _NBUF
            + [pltpu.VMEM((_CH, H2), F32)] * _NBUF
            + [
                pltpu.VMEM((_TAIL,), jnp.int32),
                pltpu.VMEM((_TAIL,), jnp.int32),
                pltpu.VMEM_SHARED((N, H2), F32),
            ]
            + [pltpu.SemaphoreType.DMA] * (3 * _NBUF)
        ),
    )


def _sc_propagate_body(ha_hbm, hb_hbm, e_hbm, zero_hbm, out_hbm, *refs):
    src_bufs = refs[0:_NBUF]
    dst_bufs = refs[_NBUF:2 * _NBUF]
    row_bufs = refs[2 * _NBUF:3 * _NBUF]
    src_t, dst_t, acc = refs[3 * _NBUF:3 * _NBUF + 3]
    sems = refs[3 * _NBUF + 3:]
    isems = sems[0:_NBUF]
    gsems = sems[_NBUF:2 * _NBUF]
    ssems = sems[2 * _NBUF:3 * _NBUF]

    c = lax.axis_index("c")
    s = lax.axis_index("s")

    pltpu.sync_copy(zero_hbm.at[pl.ds(s * _RPT, _RPT)],
                    acc.at[pl.ds(s * _RPT, _RPT)])

    @pl.when(s == NS - 1)
    def _():
        pltpu.sync_copy(zero_hbm.at[pl.ds(NS * _RPT, _RPT_REM)],
                        acc.at[pl.ds(NS * _RPT, _RPT_REM)])

    plsc.subcore_barrier()

    base0 = s * _EPT

    def h_ref_op(sv, rv, sem, wait):
        @pl.when(c == 0)
        def _():
            cp = pltpu.make_async_copy(ha_hbm.at[sv], rv, sem)
            cp.wait() if wait else cp.start()

        @pl.when(c == 1)
        def _():
            cp = pltpu.make_async_copy(hb_hbm.at[sv], rv, sem)
            cp.wait() if wait else cp.start()

    def body(k, _):
        for i in range(_NBUF):
            j = _NBUF * k + i
            sv, dv, rv = src_bufs[i], dst_bufs[i], row_bufs[i]

            @pl.when(k > 0)
            def _():
                pltpu.make_async_copy(rv, acc.at[dv], ssems[i]).wait()

            pltpu.async_copy(e_hbm.at[pl.ds(base0 + j * _CH, _CH)], sv,
                             isems[i])
            pltpu.async_copy(e_hbm.at[pl.ds(E + base0 + j * _CH, _CH)], dv,
                             isems[i])
        for i in range(_NBUF):
            j = _NBUF * k + i
            sv, dv, rv = src_bufs[i], dst_bufs[i], row_bufs[i]
            pltpu.make_async_copy(e_hbm.at[pl.ds(base0 + j * _CH, _CH)], sv,
                                  isems[i]).wait()
            pltpu.make_async_copy(e_hbm.at[pl.ds(E + base0 + j * _CH, _CH)],
                                  dv, isems[i]).wait()
            h_ref_op(sv, rv, gsems[i], wait=False)
        for i in range(_NBUF):
            sv, dv, rv = src_bufs[i], dst_bufs[i], row_bufs[i]
            h_ref_op(sv, rv, gsems[i], wait=True)
            pltpu.async_copy(rv, acc.at[dv], ssems[i], add=True)
        return 0

    lax.fori_loop(0, _FULL // _NBUF, body, 0)
    for i in range(_NBUF):
        pltpu.make_async_copy(row_bufs[i], acc.at[dst_bufs[i]],
                              ssems[i]).wait()

    if _TAIL:
        base = base0 + _FULL * _CH
        rows_t = row_bufs[0].at[pl.ds(0, _TAIL)]
        pltpu.sync_copy(e_hbm.at[pl.ds(base, _TAIL)], src_t)
        pltpu.sync_copy(e_hbm.at[pl.ds(E + base, _TAIL)], dst_t)
        h_ref_op(src_t, rows_t, gsems[0], wait=False)
        h_ref_op(src_t, rows_t, gsems[0], wait=True)
        pltpu.sync_copy(rows_t, acc.at[dst_t], add=True)

    plsc.subcore_barrier()
    pltpu.sync_copy(acc.at[pl.ds(s * _RPT, _RPT)],
                    out_hbm.at[c].at[pl.ds(s * _RPT, _RPT)])

    @pl.when(s == NS - 1)
    def _():
        pltpu.sync_copy(acc.at[pl.ds(NS * _RPT, _RPT_REM)],
                        out_hbm.at[c].at[pl.ds(NS * _RPT, _RPT_REM)])


_BM = 1000


def _norms_body(degp_ref, out_ref):
    deg = jnp.sum(degp_ref[...], axis=0)
    out_ref[...] = lax.rsqrt(jnp.maximum(deg, 1.0))


def _tc_norms(degp):
    return pl.pallas_call(
        _norms_body,
        out_shape=jax.ShapeDtypeStruct((2, N), F32),
    )(degp)


def _scale_body(f_ref, on_ref, a_ref, b_ref):
    xs = f_ref[...] * on_ref[...]
    a_ref[...] = xs[:, :H2]
    b_ref[...] = xs[:, H2:]


def _tc_scale(features, onorm):
    grid = (N // _BM,)
    return pl.pallas_call(
        _scale_body,
        grid=grid,
        in_specs=[
            pl.BlockSpec((_BM, IN_DIM), lambda i: (i, 0)),
            pl.BlockSpec((_BM, 1), lambda i: (i, 0)),
        ],
        out_specs=[
            pl.BlockSpec((_BM, H2), lambda i: (i, 0)),
            pl.BlockSpec((_BM, H2), lambda i: (i, 0)),
        ],
        out_shape=[
            jax.ShapeDtypeStruct((N, H2), F32),
            jax.ShapeDtypeStruct((N, H2), F32),
        ],
        compiler_params=pltpu.CompilerParams(
            dimension_semantics=("parallel",)),
    )(features, onorm)


def _layer1_body(a0_ref, a1_ref, w_ref, b_ref, in_ref, on_ref, ha_ref, hb_ref):
    w = w_ref[...]
    t = jnp.dot(a0_ref[0], w[:H2, :], preferred_element_type=F32)
    t += jnp.dot(a1_ref[0], w[H2:, :], preferred_element_type=F32)
    t = t * in_ref[...] + b_ref[...]
    t = jnp.maximum(t, 0.0) * on_ref[...]
    ha_ref[...] = t[:, :H2]
    hb_ref[...] = t[:, H2:]


def _tc_layer1(agg, W1, b1r, inorm, onorm):
    grid = (N // _BM,)
    return pl.pallas_call(
        _layer1_body,
        grid=grid,
        in_specs=[
            pl.BlockSpec((1, _BM, H2), lambda i: (0, i, 0)),
            pl.BlockSpec((1, _BM, H2), lambda i: (1, i, 0)),
            pl.BlockSpec((IN_DIM, H1), lambda i: (0, 0)),
            pl.BlockSpec((1, H1), lambda i: (0, 0)),
            pl.BlockSpec((_BM, 1), lambda i: (i, 0)),
            pl.BlockSpec((_BM, 1), lambda i: (i, 0)),
        ],
        out_specs=[
            pl.BlockSpec((_BM, H2), lambda i: (i, 0)),
            pl.BlockSpec((_BM, H2), lambda i: (i, 0)),
        ],
        out_shape=[
            jax.ShapeDtypeStruct((N, H2), F32),
            jax.ShapeDtypeStruct((N, H2), F32),
        ],
        compiler_params=pltpu.CompilerParams(
            dimension_semantics=("parallel",)),
    )(agg, agg, W1, b1r, inorm, onorm)


def _z_body(a0_ref, a1_ref, w_ref, b_ref, in_ref, nz_ref, z_ref):
    w = w_ref[...]
    t = jnp.dot(a0_ref[0], w[:H2, :], preferred_element_type=F32)
    t += jnp.dot(a1_ref[0], w[H2:, :], preferred_element_type=F32)
    t = t * in_ref[...] + b_ref[...]
    z_ref[...] = t[:, :H2] + nz_ref[...] * jnp.exp(t[:, H2:])


def _tc_z(aggh, W23, b23r, inorm, noise):
    grid = (N // _BM,)
    return pl.pallas_call(
        _z_body,
        grid=grid,
        in_specs=[
            pl.BlockSpec((1, _BM, H2), lambda i: (0, i, 0)),
            pl.BlockSpec((1, _BM, H2), lambda i: (1, i, 0)),
            pl.BlockSpec((H1, 2 * H2), lambda i: (0, 0)),
            pl.BlockSpec((1, 2 * H2), lambda i: (0, 0)),
            pl.BlockSpec((_BM, 1), lambda i: (i, 0)),
            pl.BlockSpec((_BM, H2), lambda i: (i, 0)),
        ],
        out_specs=pl.BlockSpec((_BM, H2), lambda i: (i, 0)),
        out_shape=jax.ShapeDtypeStruct((N, H2), F32),
        compiler_params=pltpu.CompilerParams(
            dimension_semantics=("parallel",)),
    )(aggh, aggh, W23, b23r, inorm, noise)


_DBM = 256


def _dec_body(zr_ref, zc_ref, out_ref):
    acc = lax.dot_general(zr_ref[...], zc_ref[...],
                          (((1,), (1,)), ((), ())),
                          preferred_element_type=F32)
    out_ref[...] = jax.nn.sigmoid(acc)


def _tc_decoder(z):
    grid = (pl.cdiv(N, _DBM),)
    return pl.pallas_call(
        _dec_body,
        grid=grid,
        in_specs=[
            pl.BlockSpec((_DBM, H2), lambda i: (i, 0)),
            pl.BlockSpec((N, H2), lambda i: (0, 0)),
        ],
        out_specs=pl.BlockSpec((_DBM, N), lambda i: (i, 0)),
        out_shape=jax.ShapeDtypeStruct((N, N), F32),
        compiler_params=pltpu.CompilerParams(
            dimension_semantics=("parallel",)),
    )(z, z)


@functools.cache
def _noise_const():
    with jax.ensure_compile_time_eval():
        return jax.random.normal(jax.random.key(42), (N, H2), dtype=F32)


def kernel(features, edge_index, W1, b1, W2, b2, W3, b3):
    eflat = edge_index.reshape(2 * E)

    degp = _make_sc_degrees()(eflat)
    norms = _tc_norms(degp)
    onorm = norms[0].reshape(N, 1)
    inorm = norms[1].reshape(N, 1)

    xsA, xsB = _tc_scale(features, onorm)
    zeros = jnp.zeros((N, H2), F32)
    propagate = _make_sc_propagate()
    agg = propagate(xsA, xsB, eflat, zeros)
    hA, hB = _tc_layer1(agg, W1, b1.reshape(1, H1), inorm, onorm)
    aggh = propagate(hA, hB, eflat, zeros)

    W23 = jnp.concatenate([W2, W3], axis=1)
    b23 = jnp.concatenate([b2, b3]).reshape(1, 2 * H2)
    z = _tc_z(aggh, W23, b23, inorm, _noise_const())

    return _tc_decoder(z)


# speedup vs baseline: 5.9996x; 1.0298x over previous
import functools

import jax
import jax.numpy as jnp
from jax import lax
from jax.experimental import pallas as pl
from jax.experimental.pallas import tpu as pltpu
from jax.experimental.pallas import tpu_sc as plsc

N = 10000
E = 160000
IN_DIM = 256
H1 = 256
H2 = 128

NC = 2
NS = 16
NW = NC * NS

F32 = jnp.float32


_DEG_EPT = E // NW
_DEG_FULL = _DEG_EPT // 16
_DEG_REM = _DEG_EPT - _DEG_FULL * 16

@functools.cache
def _make_sc_degrees():
    mesh = plsc.VectorSubcoreMesh(core_axis_name="c", subcore_axis_name="s",
                                  num_cores=NC, num_subcores=NS)
    return pl.kernel(
        _sc_degrees_body,
        out_type=jax.ShapeDtypeStruct((NW, 2, N), F32),
        mesh=mesh,
        compiler_params=pltpu.CompilerParams(needs_layout_passes=False),
        scratch_types=[
            pltpu.VMEM((_DEG_EPT + 16,), jnp.int32),
            pltpu.VMEM((_DEG_EPT + 16,), jnp.int32),
            pltpu.VMEM((N,), F32),
            pltpu.VMEM((N,), F32),
        ],
    )


def _sc_degrees_body(e_hbm, out_hbm, src_v, dst_v, acc_o, acc_i):
    c = lax.axis_index("c")
    s = lax.axis_index("s")
    wid = s * NC + c
    base = wid * _DEG_EPT

    zeros16 = jnp.zeros((16,), F32)

    def zero_body(i, _):
        acc_o[pl.ds(i * 16, 16)] = zeros16
        acc_i[pl.ds(i * 16, 16)] = zeros16
        return 0

    lax.fori_loop(0, N // 16, zero_body, 0)

    pltpu.sync_copy(e_hbm.at[pl.ds(base, _DEG_EPT)],
                    src_v.at[pl.ds(0, _DEG_EPT)])
    pltpu.sync_copy(e_hbm.at[pl.ds(E + base, _DEG_EPT)],
                    dst_v.at[pl.ds(0, _DEG_EPT)])

    ones16 = jnp.ones((16,), F32)

    def body(i, _):
        sv = src_v[pl.ds(i * 16, 16)]
        dv = dst_v[pl.ds(i * 16, 16)]
        plsc.addupdate_scatter(acc_o, [sv], ones16)
        plsc.addupdate_scatter(acc_i, [dv], ones16)
        return 0

    lax.fori_loop(0, _DEG_FULL, body, 0)

    if _DEG_REM:
        mask = lax.iota(jnp.int32, 16) < _DEG_REM
        sv = src_v[pl.ds(_DEG_FULL * 16, 16)]
        dv = dst_v[pl.ds(_DEG_FULL * 16, 16)]
        plsc.addupdate_scatter(acc_o, [sv], ones16, mask=mask)
        plsc.addupdate_scatter(acc_i, [dv], ones16, mask=mask)

    pltpu.sync_copy(acc_o, out_hbm.at[wid, 0])
    pltpu.sync_copy(acc_i, out_hbm.at[wid, 1])


_EPT = E // NS
_CH = 96
_FULL = _EPT // _CH
_TAIL = _EPT - _FULL * _CH
_NBUF = 4

_RPT = 624
_RPT_REM = N - NS * _RPT


@functools.cache
def _make_sc_propagate():
    mesh = plsc.VectorSubcoreMesh(core_axis_name="c", subcore_axis_name="s",
                                  num_cores=NC, num_subcores=NS)
    return pl.kernel(
        _sc_propagate_body,
        out_type=jax.ShapeDtypeStruct((NC, N, H2), F32),
        mesh=mesh,
        compiler_params=pltpu.CompilerParams(needs_layout_passes=False),
        scratch_types=(
            [pltpu.VMEM((_CH,), jnp.int32)] * _NBUF
            + [pltpu.VMEM((_CH,), jnp.int32)] * _NBUF
            + [pltpu.VMEM((_CH, H2), F32)] * _NBUF
            + [
                pltpu.VMEM((_TAIL,), jnp.int32),
                pltpu.VMEM((_TAIL,), jnp.int32),
                pltpu.VMEM_SHARED((N, H2), F32),
            ]
            + [pltpu.SemaphoreType.DMA] * (3 * _NBUF)
        ),
    )


def _sc_propagate_body(ha_hbm, hb_hbm, e_hbm, out_hbm, *refs):
    src_bufs = refs[0:_NBUF]
    dst_bufs = refs[_NBUF:2 * _NBUF]
    row_bufs = refs[2 * _NBUF:3 * _NBUF]
    src_t, dst_t, acc = refs[3 * _NBUF:3 * _NBUF + 3]
    sems = refs[3 * _NBUF + 3:]
    isems = sems[0:_NBUF]
    gsems = sems[_NBUF:2 * _NBUF]
    ssems = sems[2 * _NBUF:3 * _NBUF]

    c = lax.axis_index("c")
    s = lax.axis_index("s")

    zeros16 = jnp.zeros((16,), F32)

    def zero_body(t, _):
        row_bufs[0][t // 8, pl.ds((t % 8) * 16, 16)] = zeros16
        return 0

    lax.fori_loop(0, _CH * (H2 // 16), zero_body, 0)

    n_rep = _RPT // _CH
    rem = _RPT - n_rep * _CH

    def zcp_body(r, _):
        pltpu.sync_copy(row_bufs[0],
                        acc.at[pl.ds(s * _RPT + r * _CH, _CH)])
        return 0

    lax.fori_loop(0, n_rep, zcp_body, 0)
    if rem:
        pltpu.sync_copy(row_bufs[0].at[pl.ds(0, rem)],
                        acc.at[pl.ds(s * _RPT + n_rep * _CH, rem)])

    @pl.when(s == NS - 1)
    def _():
        pltpu.sync_copy(row_bufs[0].at[pl.ds(0, _RPT_REM)],
                        acc.at[pl.ds(NS * _RPT, _RPT_REM)])

    plsc.subcore_barrier()

    base0 = s * _EPT

    def h_ref_op(sv, rv, sem, wait):
        @pl.when(c == 0)
        def _():
            cp = pltpu.make_async_copy(ha_hbm.at[sv], rv, sem)
            cp.wait() if wait else cp.start()

        @pl.when(c == 1)
        def _():
            cp = pltpu.make_async_copy(hb_hbm.at[sv], rv, sem)
            cp.wait() if wait else cp.start()

    def body(k, _):
        for i in range(_NBUF):
            j = _NBUF * k + i
            sv, dv, rv = src_bufs[i], dst_bufs[i], row_bufs[i]

            @pl.when(k > 0)
            def _():
                pltpu.make_async_copy(rv, acc.at[dv], ssems[i]).wait()

            pltpu.async_copy(e_hbm.at[pl.ds(base0 + j * _CH, _CH)], sv,
                             isems[i])
            pltpu.async_copy(e_hbm.at[pl.ds(E + base0 + j * _CH, _CH)], dv,
                             isems[i])
        for i in range(_NBUF):
            j = _NBUF * k + i
            sv, dv, rv = src_bufs[i], dst_bufs[i], row_bufs[i]
            pltpu.make_async_copy(e_hbm.at[pl.ds(base0 + j * _CH, _CH)], sv,
                                  isems[i]).wait()
            pltpu.make_async_copy(e_hbm.at[pl.ds(E + base0 + j * _CH, _CH)],
                                  dv, isems[i]).wait()
            h_ref_op(sv, rv, gsems[i], wait=False)
        for i in range(_NBUF):
            sv, dv, rv = src_bufs[i], dst_bufs[i], row_bufs[i]
            h_ref_op(sv, rv, gsems[i], wait=True)
            pltpu.async_copy(rv, acc.at[dv], ssems[i], add=True)
        return 0

    lax.fori_loop(0, _FULL // _NBUF, body, 0)
    for i in range(_NBUF):
        pltpu.make_async_copy(row_bufs[i], acc.at[dst_bufs[i]],
                              ssems[i]).wait()

    if _TAIL:
        base = base0 + _FULL * _CH
        rows_t = row_bufs[0].at[pl.ds(0, _TAIL)]
        pltpu.sync_copy(e_hbm.at[pl.ds(base, _TAIL)], src_t)
        pltpu.sync_copy(e_hbm.at[pl.ds(E + base, _TAIL)], dst_t)
        h_ref_op(src_t, rows_t, gsems[0], wait=False)
        h_ref_op(src_t, rows_t, gsems[0], wait=True)
        pltpu.sync_copy(rows_t, acc.at[dst_t], add=True)

    plsc.subcore_barrier()
    pltpu.sync_copy(acc.at[pl.ds(s * _RPT, _RPT)],
                    out_hbm.at[c].at[pl.ds(s * _RPT, _RPT)])

    @pl.when(s == NS - 1)
    def _():
        pltpu.sync_copy(acc.at[pl.ds(NS * _RPT, _RPT_REM)],
                        out_hbm.at[c].at[pl.ds(NS * _RPT, _RPT_REM)])


_BM = 1000


def _norms_body(degp_ref, out_ref):
    deg = jnp.sum(degp_ref[...], axis=0)
    out_ref[...] = lax.rsqrt(jnp.maximum(deg, 1.0))


def _tc_norms(degp):
    return pl.pallas_call(
        _norms_body,
        out_shape=jax.ShapeDtypeStruct((2, N), F32),
    )(degp)


def _scale_body(f_ref, on_ref, a_ref, b_ref):
    xs = f_ref[...] * on_ref[...]
    a_ref[...] = xs[:, :H2]
    b_ref[...] = xs[:, H2:]


def _tc_scale(features, onorm):
    grid = (N // _BM,)
    return pl.pallas_call(
        _scale_body,
        grid=grid,
        in_specs=[
            pl.BlockSpec((_BM, IN_DIM), lambda i: (i, 0)),
            pl.BlockSpec((_BM, 1), lambda i: (i, 0)),
        ],
        out_specs=[
            pl.BlockSpec((_BM, H2), lambda i: (i, 0)),
            pl.BlockSpec((_BM, H2), lambda i: (i, 0)),
        ],
        out_shape=[
            jax.ShapeDtypeStruct((N, H2), F32),
            jax.ShapeDtypeStruct((N, H2), F32),
        ],
        compiler_params=pltpu.CompilerParams(
            dimension_semantics=("parallel",)),
    )(features, onorm)


def _layer1_body(a0_ref, a1_ref, w_ref, b_ref, in_ref, on_ref, ha_ref, hb_ref):
    w = w_ref[...]
    t = jnp.dot(a0_ref[0], w[:H2, :], preferred_element_type=F32)
    t += jnp.dot(a1_ref[0], w[H2:, :], preferred_element_type=F32)
    t = t * in_ref[...] + b_ref[...]
    t = jnp.maximum(t, 0.0) * on_ref[...]
    ha_ref[...] = t[:, :H2]
    hb_ref[...] = t[:, H2:]


def _tc_layer1(agg, W1, b1r, inorm, onorm):
    grid = (N // _BM,)
    return pl.pallas_call(
        _layer1_body,
        grid=grid,
        in_specs=[
            pl.BlockSpec((1, _BM, H2), lambda i: (0, i, 0)),
            pl.BlockSpec((1, _BM, H2), lambda i: (1, i, 0)),
            pl.BlockSpec((IN_DIM, H1), lambda i: (0, 0)),
            pl.BlockSpec((1, H1), lambda i: (0, 0)),
            pl.BlockSpec((_BM, 1), lambda i: (i, 0)),
            pl.BlockSpec((_BM, 1), lambda i: (i, 0)),
        ],
        out_specs=[
            pl.BlockSpec((_BM, H2), lambda i: (i, 0)),
            pl.BlockSpec((_BM, H2), lambda i: (i, 0)),
        ],
        out_shape=[
            jax.ShapeDtypeStruct((N, H2), F32),
            jax.ShapeDtypeStruct((N, H2), F32),
        ],
        compiler_params=pltpu.CompilerParams(
            dimension_semantics=("parallel",)),
    )(agg, agg, W1, b1r, inorm, onorm)


def _z_body(a0_ref, a1_ref, w_ref, b_ref, in_ref, nz_ref, z_ref):
    w = w_ref[...]
    t = jnp.dot(a0_ref[0], w[:H2, :], preferred_element_type=F32)
    t += jnp.dot(a1_ref[0], w[H2:, :], preferred_element_type=F32)
    t = t * in_ref[...] + b_ref[...]
    z_ref[...] = t[:, :H2] + nz_ref[...] * jnp.exp(t[:, H2:])


def _tc_z(aggh, W23, b23r, inorm, noise):
    grid = (N // _BM,)
    return pl.pallas_call(
        _z_body,
        grid=grid,
        in_specs=[
            pl.BlockSpec((1, _BM, H2), lambda i: (0, i, 0)),
            pl.BlockSpec((1, _BM, H2), lambda i: (1, i, 0)),
            pl.BlockSpec((H1, 2 * H2), lambda i: (0, 0)),
            pl.BlockSpec((1, 2 * H2), lambda i: (0, 0)),
            pl.BlockSpec((_BM, 1), lambda i: (i, 0)),
            pl.BlockSpec((_BM, H2), lambda i: (i, 0)),
        ],
        out_specs=pl.BlockSpec((_BM, H2), lambda i: (i, 0)),
        out_shape=jax.ShapeDtypeStruct((N, H2), F32),
        compiler_params=pltpu.CompilerParams(
            dimension_semantics=("parallel",)),
    )(aggh, aggh, W23, b23r, inorm, noise)


_DBM = 512


def _dec_body(zr_ref, zc_ref, out_ref):
    acc = lax.dot_general(zr_ref[...], zc_ref[...],
                          (((1,), (1,)), ((), ())),
                          preferred_element_type=F32)
    out_ref[...] = jax.nn.sigmoid(acc)


def _tc_decoder(z):
    grid = (pl.cdiv(N, _DBM),)
    return pl.pallas_call(
        _dec_body,
        grid=grid,
        in_specs=[
            pl.BlockSpec((_DBM, H2), lambda i: (i, 0)),
            pl.BlockSpec((N, H2), lambda i: (0, 0)),
        ],
        out_specs=pl.BlockSpec((_DBM, N), lambda i: (i, 0)),
        out_shape=jax.ShapeDtypeStruct((N, N), F32),
        compiler_params=pltpu.CompilerParams(
            dimension_semantics=("parallel",)),
    )(z, z)


@functools.cache
def _noise_const():
    with jax.ensure_compile_time_eval():
        return jax.random.normal(jax.random.key(42), (N, H2), dtype=F32)


def kernel(features, edge_index, W1, b1, W2, b2, W3, b3):
    eflat = edge_index.reshape(2 * E)

    degp = _make_sc_degrees()(eflat)
    norms = _tc_norms(degp)
    onorm = norms[0].reshape(N, 1)
    inorm = norms[1].reshape(N, 1)

    xsA, xsB = _tc_scale(features, onorm)
    propagate = _make_sc_propagate()
    agg = propagate(xsA, xsB, eflat)
    hA, hB = _tc_layer1(agg, W1, b1.reshape(1, H1), inorm, onorm)
    aggh = propagate(hA, hB, eflat)

    W23 = jnp.concatenate([W2, W3], axis=1)
    b23 = jnp.concatenate([b2, b3]).reshape(1, 2 * H2)
    z = _tc_z(aggh, W23, b23, inorm, _noise_const())

    return _tc_decoder(z)
